# baseline (speedup 1.0000x reference)
import functools

import jax
import jax.numpy as jnp
from jax import lax
from jax.experimental import pallas as pl
from jax.experimental.pallas import tpu as pltpu
from jax.experimental.pallas import tpu_sc as plsc

B = 4
NBUS = 10000
E = 160000
LAT = 10
DT = 0.2
NSTEP = 5

M = B * NBUS
NB = 40960
DUMMY = M
ETOT = B * E
EP = 655360
IDXROWS = EP // 128
NWORK = 32
RPT = IDXROWS // NWORK
ROWS_PER_TILE = NB // 16

GEP = EP // 8
NBG = NB // 8
GBLK = 512
BBLKG = 512

_f32 = jnp.float32


CH = 16
NCH = RPT // CH


@functools.lru_cache(maxsize=None)
def _sc_kernels():
    mesh = plsc.VectorSubcoreMesh(core_axis_name="c", subcore_axis_name="s")

    @functools.partial(
        pl.kernel,
        out_type=(
            jax.ShapeDtypeStruct((EP, 16), jnp.bfloat16),
            jax.ShapeDtypeStruct((EP, 16), jnp.bfloat16),
        ),
        mesh=mesh,
        scratch_types=[
            pltpu.VMEM((NCH, CH * 128), jnp.int32),
            pltpu.VMEM((2, CH * 128, 16), jnp.bfloat16),
            pltpu.SemaphoreType.DMA,
            pltpu.SemaphoreType.DMA,
        ],
        compiler_params=pltpu.CompilerParams(use_tc_tiling_on_sc=False),
    )
    def sc_gather(tab, idxf, idxt, outf, outt, idx_v, buf, sem0, sem1):
        wid = lax.axis_index("s") * 2 + lax.axis_index("c")
        base = wid * RPT * 128
        sems = (sem0, sem1)

        def run(idx_hbm, out_hbm):
            pltpu.sync_copy(idx_hbm.at[pl.ds(wid * NCH, NCH)], idx_v)
            d = {0: pltpu.async_copy(
                tab.at[idx_v.at[0]], buf.at[0], sems[0])}
            for k in range(NCH):
                par = k & 1
                if k + 1 < NCH:
                    d[k + 1] = pltpu.async_copy(
                        tab.at[idx_v.at[k + 1]],
                        buf.at[1 - par], sems[1 - par])
                d[k].wait()
                pltpu.sync_copy(buf.at[par],
                                out_hbm.at[pl.ds(base + k * CH * 128,
                                                 CH * 128)])

        run(idxf, outf)
        run(idxt, outt)

    @functools.partial(
        pl.kernel,
        out_type=jax.ShapeDtypeStruct((2 * NB, 16), _f32),
        mesh=mesh,
        scratch_types=[
            pltpu.VMEM((NCH, CH * 128), jnp.int32),
            pltpu.VMEM((2, CH * 128, 16), _f32),
            pltpu.VMEM_SHARED((NB, 16), _f32),
            pltpu.SemaphoreType.DMA,
            pltpu.SemaphoreType.DMA,
            pltpu.SemaphoreType.DMA,
        ],
        compiler_params=pltpu.CompilerParams(use_tc_tiling_on_sc=False),
    )
    def sc_scatter(p1, idxf, p2, idxt, zeros_hbm, out,
                   idx_v, val, acc, lsem0, lsem1, asem):
        c = lax.axis_index("c")
        s = lax.axis_index("s")
        wid = s * 2 + c
        base = wid * RPT * 128
        lsems = (lsem0, lsem1)
        pltpu.sync_copy(
            zeros_hbm.at[pl.ds(s * ROWS_PER_TILE, ROWS_PER_TILE)],
            acc.at[pl.ds(s * ROWS_PER_TILE, ROWS_PER_TILE)],
        )
        plsc.subcore_barrier()

        def run(p_hbm, idx_hbm):
            pltpu.sync_copy(idx_hbm.at[pl.ds(wid * NCH, NCH)], idx_v)
            d = {0: pltpu.async_copy(
                p_hbm.at[pl.ds(base, CH * 128)], val.at[0], lsems[0])}
            for k in range(NCH):
                par = k & 1
                if k + 1 < NCH:
                    d[k + 1] = pltpu.async_copy(
                        p_hbm.at[pl.ds(base + (k + 1) * CH * 128, CH * 128)],
                        val.at[1 - par], lsems[1 - par])
                d[k].wait()
                pltpu.async_copy(
                    val.at[par], acc.at[idx_v.at[k]],
                    asem, add=True).wait()

        run(p1, idxf)
        run(p2, idxt)
        plsc.subcore_barrier()
        pltpu.sync_copy(
            acc.at[pl.ds(s * ROWS_PER_TILE, ROWS_PER_TILE)],
            out.at[pl.ds(c * NB + s * ROWS_PER_TILE, ROWS_PER_TILE)],
        )

    return sc_gather, sc_scatter


def _mm(a, b):
    return jnp.dot(a.astype(jnp.bfloat16), b, preferred_element_type=_f32)


def _gnorm(v, g16):
    s = jnp.dot(v * v, g16, preferred_element_type=_f32)
    return v / (jnp.sqrt(s) + 1.0)


def _edge_body(hvf, hvt, he0, aij, w1f, w1t, w1e, w1a, w2, w3,
               w4p1, w4p2, w4e, b123, b4, g16, p1, p2, he0n):
    x = (
        _mm(hvf[...], w1f[...])
        + _mm(hvt[...], w1t[...])
        + _mm(he0[...], w1e[...])
        + _mm(aij[...], w1a[...])
        + b123[0:1, :]
    )
    h = jnp.tanh(x)
    h = jnp.tanh(_mm(h, w2[...]) + b123[1:2, :])
    h = jnp.tanh(_mm(h, w3[...]) + b123[2:3, :])
    p1[...] = _mm(h, w4p1[...]) + b4[0:1, :]
    p2[...] = _mm(h, w4p2[...]) + b4[1:2, :]
    d = _mm(h, w4e[...]) + b4[2:3, :]
    he0n[...] = _gnorm(he0[...].astype(_f32) + DT * d,
                       g16[...]).astype(jnp.bfloat16)


def _tc_edge(hvf, hvt, he0, aij, ew, b123, b4, g16):
    grid = (GEP // GBLK,)
    row = lambda i: (i, 0)
    full = lambda i: (0, 0)
    w1f, w1t, w1e, w1a, w2, w3, w4p1, w4p2, w4e = ew
    return pl.pallas_call(
        _edge_body,
        grid=grid,
        in_specs=[
            pl.BlockSpec((GBLK, 128), row),
            pl.BlockSpec((GBLK, 128), row),
            pl.BlockSpec((GBLK, 128), row),
            pl.BlockSpec((GBLK, 128), row),
            pl.BlockSpec((128, 256), full),
            pl.BlockSpec((128, 256), full),
            pl.BlockSpec((128, 256), full),
            pl.BlockSpec((128, 256), full),
            pl.BlockSpec((256, 256), full),
            pl.BlockSpec((256, 256), full),
            pl.BlockSpec((256, 128), full),
            pl.BlockSpec((256, 128), full),
            pl.BlockSpec((256, 128), full),
            pl.BlockSpec((8, 256), full),
            pl.BlockSpec((8, 128), full),
            pl.BlockSpec((128, 128), full),
        ],
        out_specs=[
            pl.BlockSpec((GBLK, 128), row),
            pl.BlockSpec((GBLK, 128), row),
            pl.BlockSpec((GBLK, 128), row),
        ],
        out_shape=[
            jax.ShapeDtypeStruct((GEP, 128), _f32),
            jax.ShapeDtypeStruct((GEP, 128), _f32),
            jax.ShapeDtypeStruct((GEP, 128), jnp.bfloat16),
        ],
    )(hvf, hvt, he0, aij, w1f, w1t, w1e, w1a, w2, w3, w4p1, w4p2, w4e,
      b123, b4, g16)


def _bus_mlp(hv, he1, u, bi, w1h, w1e, w1u, w1b, w2, w3, w4, b):
    x = (
        _mm(hv, w1h)
        + _mm(he1, w1e)
        + _mm(u, w1u)
        + _mm(bi, w1b)
        + b[0:1, :]
    )
    h = jnp.tanh(x)
    h = jnp.tanh(_mm(h, w2) + b[1:2, :])
    h = jnp.tanh(_mm(h, w3) + b[2:3, :])
    return _mm(h, w4) + b[3:4, :]


def _bus_body(hv, he1, u, bi, acc0, acc1,
              vw1h, vw1e, vw1u, vw1b, vw2, vw3, vw4, vb,
              ew1h, ew1e, ew1u, ew1b, ew2, ew3, ew4, eb,
              ow1h, ow1e, ow1u, ow1b, ow2, ow3, ow4, ob,
              g16, hvn, hvb, he1n, un):
    hv_ = hv[...]
    he1_ = he1[...]
    u_ = u[...]
    bi_ = bi[...]
    g = g16[...]
    d1 = _bus_mlp(hv_, he1_, u_, bi_, vw1h[...], vw1e[...], vw1u[...],
                  vw1b[...], vw2[...], vw3[...], vw4[...], vb)
    hvnew = _gnorm(hv_ + DT * d1 + acc0[...] + acc1[...], g)
    d2 = _bus_mlp(hvnew, he1_, u_, bi_, ew1h[...], ew1e[...], ew1u[...],
                  ew1b[...], ew2[...], ew3[...], ew4[...], eb)
    he1new = _gnorm(he1_ + DT * d2, g)
    d3 = _bus_mlp(hvnew, he1new, u_, bi_, ow1h[...], ow1e[...], ow1u[...],
                  ow1b[...], ow2[...], ow3[...], ow4[...], ob)
    hvn[...] = hvnew
    hvb[...] = hvnew.astype(jnp.bfloat16)
    he1n[...] = he1new
    un[...] = u_ + DT * d3


def _tc_bus(hv, he1, u, bi, acc0, acc1, wsets, g16):
    grid = (NBG // BBLKG,)
    row = lambda i: (i, 0)
    full = lambda i: (0, 0)
    wspecs = []
    wargs = []
    for ws in wsets:
        wspecs += [pl.BlockSpec((128, 128), full)] * 7 + [
            pl.BlockSpec((8, 128), full)]
        wargs += list(ws)
    return pl.pallas_call(
        _bus_body,
        grid=grid,
        in_specs=[pl.BlockSpec((BBLKG, 128), row)] * 6 + wspecs
        + [pl.BlockSpec((128, 128), full)],
        out_specs=[pl.BlockSpec((BBLKG, 128), row)] * 4,
        out_shape=[
            jax.ShapeDtypeStruct((NBG, 128), _f32),
            jax.ShapeDtypeStruct((NBG, 128), jnp.bfloat16),
            jax.ShapeDtypeStruct((NBG, 128), _f32),
            jax.ShapeDtypeStruct((NBG, 128), _f32),
        ],
    )(hv, he1, u, bi, acc0, acc1, *wargs, g16)


def _pack_edge_weights(params):
    eye8 = jnp.eye(8, dtype=_f32)
    bf = jnp.bfloat16

    def base16(rows, sl):
        out = jnp.zeros((16, 32), _f32)
        for i, name in enumerate(("pv00", "pv01", "pe0")):
            W1 = params[name][0][0]
            out = out.at[0:rows, i * 10:i * 10 + 10].set(W1[sl])
        return out

    w1f = base16(10, slice(1, 11))
    w1t = base16(10, slice(11, 21))
    w1e = base16(10, slice(21, 31))
    w1a = base16(8, slice(31, 39))

    def blockdiag(layer):
        out = jnp.zeros((32, 32), _f32)
        for i, name in enumerate(("pv00", "pv01", "pe0")):
            out = out.at[i * 10:i * 10 + 10, i * 10:i * 10 + 10].set(
                params[name][layer][0])
        return out

    w2 = blockdiag(1)
    w3 = blockdiag(2)

    def w4base(which):
        out = jnp.zeros((32, 16), _f32)
        i = ("pv00", "pv01", "pe0").index(which)
        return out.at[i * 10:i * 10 + 10, 0:10].set(params[which][3][0])

    big = lambda m: jnp.kron(eye8, m).astype(bf)
    ws = (big(w1f), big(w1t), big(w1e), big(w1a), big(w2), big(w3),
          big(w4base("pv00")), big(w4base("pv01")), big(w4base("pe0")))

    def bvec(layer):
        out = jnp.zeros((32,), _f32)
        for i, name in enumerate(("pv00", "pv01", "pe0")):
            out = out.at[i * 10:i * 10 + 10].set(params[name][layer][1])
        return out

    trow = jnp.zeros((32,), _f32)
    for i, name in enumerate(("pv00", "pv01", "pe0")):
        trow = trow.at[i * 10:i * 10 + 10].set(params[name][0][0][0])
    b1 = bvec(0)
    b2 = bvec(1)
    b3 = bvec(2)
    b4p1 = jnp.zeros((16,), _f32).at[0:10].set(params["pv00"][3][1])
    b4p2 = jnp.zeros((16,), _f32).at[0:10].set(params["pv01"][3][1])
    b4e = jnp.zeros((16,), _f32).at[0:10].set(params["pe0"][3][1])
    b4 = jnp.concatenate([
        jnp.stack([jnp.tile(b4p1, 8), jnp.tile(b4p2, 8), jnp.tile(b4e, 8)]),
        jnp.zeros((5, 128), _f32),
    ])
    return ws, (trow, b1, b2, b3), b4


def _edge_bias(trow, b1, b2, b3, t):
    rows = jnp.stack([jnp.tile(b1 + t * trow, 8), jnp.tile(b2, 8),
                      jnp.tile(b3, 8)])
    return jnp.concatenate([rows, jnp.zeros((5, 256), _f32)])


def _pack_bus_weights(p, d_out):
    eye8 = jnp.eye(8, dtype=_f32)
    bf = jnp.bfloat16
    W1 = p[0][0]

    def base(rows, sl):
        return jnp.zeros((16, 16), _f32).at[0:rows, 0:10].set(W1[sl])

    w1h = base(10, slice(1, 11))
    w1e = base(10, slice(11, 21))
    w1u = base(2, slice(21, 23))
    w1b = base(10, slice(23, 33))
    w2 = jnp.zeros((16, 16), _f32).at[0:10, 0:10].set(p[1][0])
    w3 = jnp.zeros((16, 16), _f32).at[0:10, 0:10].set(p[2][0])
    w4 = jnp.zeros((16, 16), _f32).at[0:10, 0:d_out].set(p[3][0])
    big = lambda m: jnp.kron(eye8, m).astype(bf)
    ws = (big(w1h), big(w1e), big(w1u), big(w1b), big(w2), big(w3), big(w4))
    pad16 = lambda v, n: jnp.zeros((16,), _f32).at[0:n].set(v)
    trow = pad16(W1[0], 10)
    bs = (pad16(p[0][1], 10), pad16(p[1][1], 10), pad16(p[2][1], 10),
          pad16(p[3][1], d_out))
    return ws, (trow, bs)


def _bus_bias(trow, bs, t):
    rows = jnp.stack([jnp.tile(bs[0] + t * trow, 8), jnp.tile(bs[1], 8),
                      jnp.tile(bs[2], 8), jnp.tile(bs[3], 8)])
    return jnp.concatenate([rows, jnp.zeros((4, 128), _f32)])


def kernel(A_flat, B_flat, A0, params):
    a_ij = A_flat.reshape(B * E, 8)
    a_ij = jnp.concatenate([a_ij, jnp.zeros((B * E, 8), _f32)], axis=1)
    a_ij = jnp.concatenate([a_ij, jnp.zeros((EP - ETOT, 16), _f32)], axis=0)
    a_ij = a_ij.reshape(GEP, 128).astype(jnp.bfloat16)
    b_i = B_flat.reshape(M, 10)
    b_i = jnp.concatenate(
        [
            jnp.concatenate([b_i, jnp.zeros((M, 6), _f32)], axis=1),
            jnp.zeros((NB - M, 16), _f32),
        ],
        axis=0,
    ).reshape(NBG, 128)

    boff = (jnp.arange(B, dtype=jnp.int32) * NBUS)[:, None]
    idxf = (A0[:, :, 0].astype(jnp.int32) + boff).reshape(ETOT)
    idxt = (A0[:, :, 1].astype(jnp.int32) + boff).reshape(ETOT)
    pad = jnp.full((EP - ETOT,), DUMMY, jnp.int32)
    idxf2 = jnp.concatenate([idxf, pad]).reshape(IDXROWS // CH, CH * 128)
    idxt2 = jnp.concatenate([idxt, pad]).reshape(IDXROWS // CH, CH * 128)

    ew, ebp, eb4 = _pack_edge_weights(params)
    vw, vbp = _pack_bus_weights(params["pv10"], 10)
    pw, pbp = _pack_bus_weights(params["pe1"], 10)
    ow, obp = _pack_bus_weights(params["pout1"], 2)
    g16 = jnp.kron(jnp.eye(8, dtype=_f32), jnp.ones((16, 16), _f32))

    hv = jnp.zeros((NBG, 128), _f32)
    hv_bf = jnp.zeros((NB, 16), jnp.bfloat16)
    he0 = jnp.zeros((GEP, 128), jnp.bfloat16)
    he1 = jnp.zeros((NBG, 128), _f32)
    u = jnp.zeros((NBG, 128), _f32).at[:, 0::16].set(1.0)
    zeros_nb = jnp.zeros((NB, 16), _f32)
    sc_gather, sc_scatter = _sc_kernels()

    for step in range(NSTEP):
        t = DT * step
        ebias = _edge_bias(*ebp, t)
        vbias = _bus_bias(*vbp, t)
        pbias = _bus_bias(*pbp, t)
        obias = _bus_bias(*obp, t)

        hvf, hvt = sc_gather(hv_bf, idxf2, idxt2)
        p1, p2, he0 = _tc_edge(hvf.reshape(GEP, 128), hvt.reshape(GEP, 128),
                               he0, a_ij, ew, ebias, eb4, g16)
        accs = sc_scatter(p1.reshape(EP, 16), idxf2,
                          p2.reshape(EP, 16), idxt2, zeros_nb)
        acc0 = accs[:NB].reshape(NBG, 128)
        acc1 = accs[NB:].reshape(NBG, 128)
        hv, hvb, he1, u = _tc_bus(
            hv, he1, u, b_i, acc0, acc1,
            [(*vw, vbias), (*pw, pbias), (*ow, obias)], g16,
        )
        hv_bf = hvb.reshape(NB, 16)

    return u.reshape(NB, 16)[:M, 0:2].reshape(B, NBUS, 2)


# speedup vs baseline: 1.0845x; 1.0845x over previous
import functools

import jax
import jax.numpy as jnp
from jax import lax
from jax.experimental import pallas as pl
from jax.experimental.pallas import tpu as pltpu
from jax.experimental.pallas import tpu_sc as plsc

B = 4
NBUS = 10000
E = 160000
LAT = 10
DT = 0.2
NSTEP = 5

M = B * NBUS
NB = 40960
DUMMY = M
ETOT = B * E
EP = 655360
IDXROWS = EP // 128
NWORK = 32
RPT = IDXROWS // NWORK
ROWS_PER_TILE = NB // 16

GEP = EP // 8
NBG = NB // 8
GBLK = 512
BBLKG = 512

_f32 = jnp.float32


CH = 16
EP_H = EP // 2
GEP_H = GEP // 2
IDXR_H = IDXROWS // 2
RPT_H = IDXR_H // NWORK
NCH_H = RPT_H // CH


@functools.lru_cache(maxsize=None)
def _sc_kernels():
    mesh = plsc.VectorSubcoreMesh(core_axis_name="c", subcore_axis_name="s")

    @functools.partial(
        pl.kernel,
        out_type=(
            jax.ShapeDtypeStruct((EP_H, 16), _f32),
            jax.ShapeDtypeStruct((EP_H, 16), _f32),
        ),
        mesh=mesh,
        scratch_types=[
            pltpu.VMEM((NCH_H, CH * 128), jnp.int32),
            pltpu.VMEM((2, CH * 128, 16), _f32),
            pltpu.SemaphoreType.DMA,
            pltpu.SemaphoreType.DMA,
        ],
        compiler_params=pltpu.CompilerParams(use_tc_tiling_on_sc=False),
    )
    def sc_gather(tab, idxf, idxt, outf, outt, idx_v, buf, sem0, sem1):
        wid = lax.axis_index("s") * 2 + lax.axis_index("c")
        base = wid * RPT_H * 128
        sems = (sem0, sem1)

        def run(idx_hbm, out_hbm):
            pltpu.sync_copy(idx_hbm.at[pl.ds(wid * NCH_H, NCH_H)], idx_v)
            d = {0: pltpu.async_copy(
                tab.at[idx_v.at[0]], buf.at[0], sems[0])}
            for k in range(NCH_H):
                par = k & 1
                if k + 1 < NCH_H:
                    d[k + 1] = pltpu.async_copy(
                        tab.at[idx_v.at[k + 1]],
                        buf.at[1 - par], sems[1 - par])
                d[k].wait()
                pltpu.sync_copy(buf.at[par],
                                out_hbm.at[pl.ds(base + k * CH * 128,
                                                 CH * 128)])

        run(idxf, outf)
        run(idxt, outt)

    @functools.partial(
        pl.kernel,
        out_type=jax.ShapeDtypeStruct((2 * NB, 16), _f32),
        mesh=mesh,
        scratch_types=[
            pltpu.VMEM((NCH_H, CH * 128), jnp.int32),
            pltpu.VMEM((2, CH * 128, 16), _f32),
            pltpu.VMEM_SHARED((NB, 16), _f32),
            pltpu.SemaphoreType.DMA,
            pltpu.SemaphoreType.DMA,
            pltpu.SemaphoreType.DMA,
        ],
        compiler_params=pltpu.CompilerParams(use_tc_tiling_on_sc=False),
    )
    def sc_scatter(p1, idxf, p2, idxt, zeros_hbm, out,
                   idx_v, val, acc, lsem0, lsem1, asem):
        c = lax.axis_index("c")
        s = lax.axis_index("s")
        wid = s * 2 + c
        base = wid * RPT_H * 128
        lsems = (lsem0, lsem1)
        pltpu.sync_copy(
            zeros_hbm.at[pl.ds(s * ROWS_PER_TILE, ROWS_PER_TILE)],
            acc.at[pl.ds(s * ROWS_PER_TILE, ROWS_PER_TILE)],
        )
        plsc.subcore_barrier()

        def run(p_hbm, idx_hbm):
            pltpu.sync_copy(idx_hbm.at[pl.ds(wid * NCH_H, NCH_H)], idx_v)
            d = {0: pltpu.async_copy(
                p_hbm.at[pl.ds(base, CH * 128)], val.at[0], lsems[0])}
            for k in range(NCH_H):
                par = k & 1
                if k + 1 < NCH_H:
                    d[k + 1] = pltpu.async_copy(
                        p_hbm.at[pl.ds(base + (k + 1) * CH * 128, CH * 128)],
                        val.at[1 - par], lsems[1 - par])
                d[k].wait()
                pltpu.async_copy(
                    val.at[par], acc.at[idx_v.at[k]],
                    asem, add=True).wait()

        run(p1, idxf)
        run(p2, idxt)
        plsc.subcore_barrier()
        pltpu.sync_copy(
            acc.at[pl.ds(s * ROWS_PER_TILE, ROWS_PER_TILE)],
            out.at[pl.ds(c * NB + s * ROWS_PER_TILE, ROWS_PER_TILE)],
        )

    return sc_gather, sc_scatter


def _mm(a, b):
    return jnp.dot(a.astype(jnp.bfloat16), b, preferred_element_type=_f32)


def _gnorm(v, g16):
    s = jnp.dot(v * v, g16, preferred_element_type=_f32)
    return v / (jnp.sqrt(s) + 1.0)


def _edge_body(hvf, hvt, he0, aij, w1f, w1t, w1e, w1a, w2, w3,
               w4p1, w4p2, w4e, b123, b4, g16, p1, p2, he0n):
    x = (
        _mm(hvf[...], w1f[...])
        + _mm(hvt[...], w1t[...])
        + _mm(he0[...], w1e[...])
        + _mm(aij[...], w1a[...])
        + b123[0:1, :]
    )
    h = jnp.tanh(x)
    h = jnp.tanh(_mm(h, w2[...]) + b123[1:2, :])
    h = jnp.tanh(_mm(h, w3[...]) + b123[2:3, :])
    p1[...] = _mm(h, w4p1[...]) + b4[0:1, :]
    p2[...] = _mm(h, w4p2[...]) + b4[1:2, :]
    d = _mm(h, w4e[...]) + b4[2:3, :]
    he0n[...] = _gnorm(he0[...] + DT * d, g16[...])


def _tc_edge(hvf, hvt, he0, aij, ew, b123, b4, g16):
    grid = (GEP_H // GBLK,)
    row = lambda i: (i, 0)
    full = lambda i: (0, 0)
    w1f, w1t, w1e, w1a, w2, w3, w4p1, w4p2, w4e = ew
    return pl.pallas_call(
        _edge_body,
        grid=grid,
        in_specs=[
            pl.BlockSpec((GBLK, 128), row),
            pl.BlockSpec((GBLK, 128), row),
            pl.BlockSpec((GBLK, 128), row),
            pl.BlockSpec((GBLK, 128), row),
            pl.BlockSpec((128, 256), full),
            pl.BlockSpec((128, 256), full),
            pl.BlockSpec((128, 256), full),
            pl.BlockSpec((128, 256), full),
            pl.BlockSpec((256, 256), full),
            pl.BlockSpec((256, 256), full),
            pl.BlockSpec((256, 128), full),
            pl.BlockSpec((256, 128), full),
            pl.BlockSpec((256, 128), full),
            pl.BlockSpec((8, 256), full),
            pl.BlockSpec((8, 128), full),
            pl.BlockSpec((128, 128), full),
        ],
        out_specs=[
            pl.BlockSpec((GBLK, 128), row),
            pl.BlockSpec((GBLK, 128), row),
            pl.BlockSpec((GBLK, 128), row),
        ],
        out_shape=[
            jax.ShapeDtypeStruct((GEP_H, 128), _f32),
            jax.ShapeDtypeStruct((GEP_H, 128), _f32),
            jax.ShapeDtypeStruct((GEP_H, 128), _f32),
        ],
    )(hvf, hvt, he0, aij, w1f, w1t, w1e, w1a, w2, w3, w4p1, w4p2, w4e,
      b123, b4, g16)


def _bus_mlp(hv, he1, u, bi, w1h, w1e, w1u, w1b, w2, w3, w4, b):
    x = (
        _mm(hv, w1h)
        + _mm(he1, w1e)
        + _mm(u, w1u)
        + _mm(bi, w1b)
        + b[0:1, :]
    )
    h = jnp.tanh(x)
    h = jnp.tanh(_mm(h, w2) + b[1:2, :])
    h = jnp.tanh(_mm(h, w3) + b[2:3, :])
    return _mm(h, w4) + b[3:4, :]


def _bus_body(hv, he1, u, bi, acc0, acc1, acc2, acc3,
              vw1h, vw1e, vw1u, vw1b, vw2, vw3, vw4, vb,
              ew1h, ew1e, ew1u, ew1b, ew2, ew3, ew4, eb,
              ow1h, ow1e, ow1u, ow1b, ow2, ow3, ow4, ob,
              g16, hvn, he1n, un):
    hv_ = hv[...]
    he1_ = he1[...]
    u_ = u[...]
    bi_ = bi[...]
    g = g16[...]
    d1 = _bus_mlp(hv_, he1_, u_, bi_, vw1h[...], vw1e[...], vw1u[...],
                  vw1b[...], vw2[...], vw3[...], vw4[...], vb)
    hvnew = _gnorm(hv_ + DT * d1 + (acc0[...] + acc1[...])
                   + (acc2[...] + acc3[...]), g)
    d2 = _bus_mlp(hvnew, he1_, u_, bi_, ew1h[...], ew1e[...], ew1u[...],
                  ew1b[...], ew2[...], ew3[...], ew4[...], eb)
    he1new = _gnorm(he1_ + DT * d2, g)
    d3 = _bus_mlp(hvnew, he1new, u_, bi_, ow1h[...], ow1e[...], ow1u[...],
                  ow1b[...], ow2[...], ow3[...], ow4[...], ob)
    hvn[...] = hvnew
    he1n[...] = he1new
    un[...] = u_ + DT * d3


def _tc_bus(hv, he1, u, bi, accs4, wsets, g16):
    grid = (NBG // BBLKG,)
    row = lambda i: (i, 0)
    full = lambda i: (0, 0)
    wspecs = []
    wargs = []
    for ws in wsets:
        wspecs += [pl.BlockSpec((128, 128), full)] * 7 + [
            pl.BlockSpec((8, 128), full)]
        wargs += list(ws)
    return pl.pallas_call(
        _bus_body,
        grid=grid,
        in_specs=[pl.BlockSpec((BBLKG, 128), row)] * 8 + wspecs
        + [pl.BlockSpec((128, 128), full)],
        out_specs=[pl.BlockSpec((BBLKG, 128), row)] * 3,
        out_shape=[
            jax.ShapeDtypeStruct((NBG, 128), _f32),
            jax.ShapeDtypeStruct((NBG, 128), _f32),
            jax.ShapeDtypeStruct((NBG, 128), _f32),
        ],
    )(hv, he1, u, bi, *accs4, *wargs, g16)


def _pack_edge_weights(params):
    eye8 = jnp.eye(8, dtype=_f32)
    bf = jnp.bfloat16

    def base16(rows, sl):
        out = jnp.zeros((16, 32), _f32)
        for i, name in enumerate(("pv00", "pv01", "pe0")):
            W1 = params[name][0][0]
            out = out.at[0:rows, i * 10:i * 10 + 10].set(W1[sl])
        return out

    w1f = base16(10, slice(1, 11))
    w1t = base16(10, slice(11, 21))
    w1e = base16(10, slice(21, 31))
    w1a = base16(8, slice(31, 39))

    def blockdiag(layer):
        out = jnp.zeros((32, 32), _f32)
        for i, name in enumerate(("pv00", "pv01", "pe0")):
            out = out.at[i * 10:i * 10 + 10, i * 10:i * 10 + 10].set(
                params[name][layer][0])
        return out

    w2 = blockdiag(1)
    w3 = blockdiag(2)

    def w4base(which):
        out = jnp.zeros((32, 16), _f32)
        i = ("pv00", "pv01", "pe0").index(which)
        return out.at[i * 10:i * 10 + 10, 0:10].set(params[which][3][0])

    big = lambda m: jnp.kron(eye8, m).astype(bf)
    ws = (big(w1f), big(w1t), big(w1e), big(w1a), big(w2), big(w3),
          big(w4base("pv00")), big(w4base("pv01")), big(w4base("pe0")))

    def bvec(layer):
        out = jnp.zeros((32,), _f32)
        for i, name in enumerate(("pv00", "pv01", "pe0")):
            out = out.at[i * 10:i * 10 + 10].set(params[name][layer][1])
        return out

    trow = jnp.zeros((32,), _f32)
    for i, name in enumerate(("pv00", "pv01", "pe0")):
        trow = trow.at[i * 10:i * 10 + 10].set(params[name][0][0][0])
    b1 = bvec(0)
    b2 = bvec(1)
    b3 = bvec(2)
    b4p1 = jnp.zeros((16,), _f32).at[0:10].set(params["pv00"][3][1])
    b4p2 = jnp.zeros((16,), _f32).at[0:10].set(params["pv01"][3][1])
    b4e = jnp.zeros((16,), _f32).at[0:10].set(params["pe0"][3][1])
    b4 = jnp.concatenate([
        jnp.stack([jnp.tile(b4p1, 8), jnp.tile(b4p2, 8), jnp.tile(b4e, 8)]),
        jnp.zeros((5, 128), _f32),
    ])
    return ws, (trow, b1, b2, b3), b4


def _edge_bias(trow, b1, b2, b3, t):
    rows = jnp.stack([jnp.tile(b1 + t * trow, 8), jnp.tile(b2, 8),
                      jnp.tile(b3, 8)])
    return jnp.concatenate([rows, jnp.zeros((5, 256), _f32)])


def _pack_bus_weights(p, d_out):
    eye8 = jnp.eye(8, dtype=_f32)
    bf = jnp.bfloat16
    W1 = p[0][0]

    def base(rows, sl):
        return jnp.zeros((16, 16), _f32).at[0:rows, 0:10].set(W1[sl])

    w1h = base(10, slice(1, 11))
    w1e = base(10, slice(11, 21))
    w1u = base(2, slice(21, 23))
    w1b = base(10, slice(23, 33))
    w2 = jnp.zeros((16, 16), _f32).at[0:10, 0:10].set(p[1][0])
    w3 = jnp.zeros((16, 16), _f32).at[0:10, 0:10].set(p[2][0])
    w4 = jnp.zeros((16, 16), _f32).at[0:10, 0:d_out].set(p[3][0])
    big = lambda m: jnp.kron(eye8, m).astype(bf)
    ws = (big(w1h), big(w1e), big(w1u), big(w1b), big(w2), big(w3), big(w4))
    pad16 = lambda v, n: jnp.zeros((16,), _f32).at[0:n].set(v)
    trow = pad16(W1[0], 10)
    bs = (pad16(p[0][1], 10), pad16(p[1][1], 10), pad16(p[2][1], 10),
          pad16(p[3][1], d_out))
    return ws, (trow, bs)


def _bus_bias(trow, bs, t):
    rows = jnp.stack([jnp.tile(bs[0] + t * trow, 8), jnp.tile(bs[1], 8),
                      jnp.tile(bs[2], 8), jnp.tile(bs[3], 8)])
    return jnp.concatenate([rows, jnp.zeros((4, 128), _f32)])


def kernel(A_flat, B_flat, A0, params):
    a_ij = A_flat.reshape(B * E, 8)
    a_ij = jnp.concatenate([a_ij, jnp.zeros((B * E, 8), _f32)], axis=1)
    a_ij = jnp.concatenate([a_ij, jnp.zeros((EP - ETOT, 16), _f32)], axis=0)
    a_ij = a_ij.reshape(GEP, 128)
    aijA, aijB = a_ij[:GEP_H], a_ij[GEP_H:]
    b_i = B_flat.reshape(M, 10)
    b_i = jnp.concatenate(
        [
            jnp.concatenate([b_i, jnp.zeros((M, 6), _f32)], axis=1),
            jnp.zeros((NB - M, 16), _f32),
        ],
        axis=0,
    ).reshape(NBG, 128)

    boff = (jnp.arange(B, dtype=jnp.int32) * NBUS)[:, None]
    idxf = (A0[:, :, 0].astype(jnp.int32) + boff).reshape(ETOT)
    idxt = (A0[:, :, 1].astype(jnp.int32) + boff).reshape(ETOT)
    pad = jnp.full((EP - ETOT,), DUMMY, jnp.int32)
    idxf2 = jnp.concatenate([idxf, pad]).reshape(IDXROWS // CH, CH * 128)
    idxt2 = jnp.concatenate([idxt, pad]).reshape(IDXROWS // CH, CH * 128)
    nh = IDXR_H // CH
    idxfA, idxfB = idxf2[:nh], idxf2[nh:]
    idxtA, idxtB = idxt2[:nh], idxt2[nh:]

    ew, ebp, eb4 = _pack_edge_weights(params)
    vw, vbp = _pack_bus_weights(params["pv10"], 10)
    pw, pbp = _pack_bus_weights(params["pe1"], 10)
    ow, obp = _pack_bus_weights(params["pout1"], 2)
    g16 = jnp.kron(jnp.eye(8, dtype=_f32), jnp.ones((16, 16), _f32))

    hv = jnp.zeros((NB, 16), _f32)
    he0A = jnp.zeros((GEP_H, 128), _f32)
    he0B = jnp.zeros((GEP_H, 128), _f32)
    he1 = jnp.zeros((NBG, 128), _f32)
    u = jnp.zeros((NBG, 128), _f32).at[:, 0::16].set(1.0)
    zeros_nb = jnp.zeros((NB, 16), _f32)
    sc_gather, sc_scatter = _sc_kernels()

    for step in range(NSTEP):
        t = DT * step
        ebias = _edge_bias(*ebp, t)
        vbias = _bus_bias(*vbp, t)
        pbias = _bus_bias(*pbp, t)
        obias = _bus_bias(*obp, t)

        hvfA, hvtA = sc_gather(hv, idxfA, idxtA)
        hvfB, hvtB = sc_gather(hv, idxfB, idxtB)
        p1A, p2A, he0A = _tc_edge(hvfA.reshape(GEP_H, 128),
                                  hvtA.reshape(GEP_H, 128),
                                  he0A, aijA, ew, ebias, eb4, g16)
        accsA = sc_scatter(p1A.reshape(EP_H, 16), idxfA,
                           p2A.reshape(EP_H, 16), idxtA, zeros_nb)
        p1B, p2B, he0B = _tc_edge(hvfB.reshape(GEP_H, 128),
                                  hvtB.reshape(GEP_H, 128),
                                  he0B, aijB, ew, ebias, eb4, g16)
        accsB = sc_scatter(p1B.reshape(EP_H, 16), idxfB,
                           p2B.reshape(EP_H, 16), idxtB, zeros_nb)
        accs4 = [accsA[:NB].reshape(NBG, 128), accsA[NB:].reshape(NBG, 128),
                 accsB[:NB].reshape(NBG, 128), accsB[NB:].reshape(NBG, 128)]
        hvg, he1, u = _tc_bus(
            hv.reshape(NBG, 128), he1, u, b_i, accs4,
            [(*vw, vbias), (*pw, pbias), (*ow, obias)], g16,
        )
        hv = hvg.reshape(NB, 16)

    return u.reshape(NB, 16)[:M, 0:2].reshape(B, NBUS, 2)


# speedup vs baseline: 1.2305x; 1.1346x over previous
import functools

import jax
import jax.numpy as jnp
from jax import lax
from jax.experimental import pallas as pl
from jax.experimental.pallas import tpu as pltpu
from jax.experimental.pallas import tpu_sc as plsc

B = 4
NBUS = 10000
E = 160000
LAT = 10
DT = 0.2
NSTEP = 5

M = B * NBUS
NB = 40960
DUMMY = M
ETOT = B * E
EP = 655360
IDXROWS = EP // 128
NWORK = 32
RPT = IDXROWS // NWORK
ROWS_PER_TILE = NB // 16

GEP = EP // 8
NBG = NB // 8
GBLK = 512
BBLKG = 512

_f32 = jnp.float32


CH = 16
EP_H = EP // 2
GEP_H = GEP // 2
IDXR_H = IDXROWS // 2
RPT_H = IDXR_H // NWORK
NCH_H = RPT_H // CH


@functools.lru_cache(maxsize=None)
def _sc_kernels():
    mesh = plsc.VectorSubcoreMesh(core_axis_name="c", subcore_axis_name="s")

    @functools.partial(
        pl.kernel,
        out_type=(
            jax.ShapeDtypeStruct((EP_H, 16), _f32),
            jax.ShapeDtypeStruct((EP_H, 16), _f32),
        ),
        mesh=mesh,
        scratch_types=[
            pltpu.VMEM((NCH_H, CH * 128), jnp.int32),
            pltpu.VMEM((2, CH * 128, 16), _f32),
            pltpu.VMEM_SHARED((NB, 16), _f32),
            pltpu.SemaphoreType.DMA,
            pltpu.SemaphoreType.DMA,
        ],
        compiler_params=pltpu.CompilerParams(use_tc_tiling_on_sc=False),
    )
    def sc_gather(tab, idxf, idxt, outf, outt, idx_v, buf, tabsh, sem0, sem1):
        c = lax.axis_index("c")
        sub = lax.axis_index("s")
        wid = sub * 2 + c
        base = wid * RPT_H * 128
        sems = (sem0, sem1)
        pltpu.sync_copy(
            tab.at[pl.ds(sub * ROWS_PER_TILE, ROWS_PER_TILE)],
            tabsh.at[pl.ds(sub * ROWS_PER_TILE, ROWS_PER_TILE)],
        )
        plsc.subcore_barrier()

        def run(idx_hbm, out_hbm):
            pltpu.sync_copy(idx_hbm.at[pl.ds(wid * NCH_H, NCH_H)], idx_v)
            d = {0: pltpu.async_copy(
                tabsh.at[idx_v.at[0]], buf.at[0], sems[0])}
            for k in range(NCH_H):
                par = k & 1
                if k + 1 < NCH_H:
                    d[k + 1] = pltpu.async_copy(
                        tabsh.at[idx_v.at[k + 1]],
                        buf.at[1 - par], sems[1 - par])
                d[k].wait()
                pltpu.sync_copy(buf.at[par],
                                out_hbm.at[pl.ds(base + k * CH * 128,
                                                 CH * 128)])

        run(idxf, outf)
        run(idxt, outt)

    @functools.partial(
        pl.kernel,
        out_type=jax.ShapeDtypeStruct((2 * NB, 16), _f32),
        mesh=mesh,
        scratch_types=[
            pltpu.VMEM((NCH_H, CH * 128), jnp.int32),
            pltpu.VMEM((2, CH * 128, 16), _f32),
            pltpu.VMEM_SHARED((NB, 16), _f32),
            pltpu.SemaphoreType.DMA,
            pltpu.SemaphoreType.DMA,
            pltpu.SemaphoreType.DMA,
        ],
        compiler_params=pltpu.CompilerParams(use_tc_tiling_on_sc=False),
    )
    def sc_scatter(p1, idxf, p2, idxt, zeros_hbm, out,
                   idx_v, val, acc, lsem0, lsem1, asem):
        c = lax.axis_index("c")
        s = lax.axis_index("s")
        wid = s * 2 + c
        base = wid * RPT_H * 128
        lsems = (lsem0, lsem1)
        pltpu.sync_copy(
            zeros_hbm.at[pl.ds(s * ROWS_PER_TILE, ROWS_PER_TILE)],
            acc.at[pl.ds(s * ROWS_PER_TILE, ROWS_PER_TILE)],
        )
        plsc.subcore_barrier()

        def run(p_hbm, idx_hbm):
            pltpu.sync_copy(idx_hbm.at[pl.ds(wid * NCH_H, NCH_H)], idx_v)
            d = {0: pltpu.async_copy(
                p_hbm.at[pl.ds(base, CH * 128)], val.at[0], lsems[0])}
            for k in range(NCH_H):
                par = k & 1
                if k + 1 < NCH_H:
                    d[k + 1] = pltpu.async_copy(
                        p_hbm.at[pl.ds(base + (k + 1) * CH * 128, CH * 128)],
                        val.at[1 - par], lsems[1 - par])
                d[k].wait()
                pltpu.async_copy(
                    val.at[par], acc.at[idx_v.at[k]],
                    asem, add=True).wait()

        run(p1, idxf)
        run(p2, idxt)
        plsc.subcore_barrier()
        pltpu.sync_copy(
            acc.at[pl.ds(s * ROWS_PER_TILE, ROWS_PER_TILE)],
            out.at[pl.ds(c * NB + s * ROWS_PER_TILE, ROWS_PER_TILE)],
        )

    return sc_gather, sc_scatter


def _mm(a, b):
    return jnp.dot(a.astype(jnp.bfloat16), b, preferred_element_type=_f32)


def _gnorm(v, g16):
    s = jnp.dot(v * v, g16, preferred_element_type=_f32)
    return v / (jnp.sqrt(s) + 1.0)


def _edge_body(hvf, hvt, he0, aij, w1f, w1t, w1e, w1a, w2, w3,
               w4p1, w4p2, w4e, b123, b4, g16, p1, p2, he0n):
    x = (
        _mm(hvf[...], w1f[...])
        + _mm(hvt[...], w1t[...])
        + _mm(he0[...], w1e[...])
        + _mm(aij[...], w1a[...])
        + b123[0:1, :]
    )
    h = jnp.tanh(x)
    h = jnp.tanh(_mm(h, w2[...]) + b123[1:2, :])
    h = jnp.tanh(_mm(h, w3[...]) + b123[2:3, :])
    p1[...] = _mm(h, w4p1[...]) + b4[0:1, :]
    p2[...] = _mm(h, w4p2[...]) + b4[1:2, :]
    d = _mm(h, w4e[...]) + b4[2:3, :]
    he0n[...] = _gnorm(he0[...] + DT * d, g16[...])


def _tc_edge(hvf, hvt, he0, aij, ew, b123, b4, g16):
    grid = (GEP_H // GBLK,)
    row = lambda i: (i, 0)
    full = lambda i: (0, 0)
    w1f, w1t, w1e, w1a, w2, w3, w4p1, w4p2, w4e = ew
    return pl.pallas_call(
        _edge_body,
        grid=grid,
        in_specs=[
            pl.BlockSpec((GBLK, 128), row),
            pl.BlockSpec((GBLK, 128), row),
            pl.BlockSpec((GBLK, 128), row),
            pl.BlockSpec((GBLK, 128), row),
            pl.BlockSpec((128, 256), full),
            pl.BlockSpec((128, 256), full),
            pl.BlockSpec((128, 256), full),
            pl.BlockSpec((128, 256), full),
            pl.BlockSpec((256, 256), full),
            pl.BlockSpec((256, 256), full),
            pl.BlockSpec((256, 128), full),
            pl.BlockSpec((256, 128), full),
            pl.BlockSpec((256, 128), full),
            pl.BlockSpec((8, 256), full),
            pl.BlockSpec((8, 128), full),
            pl.BlockSpec((128, 128), full),
        ],
        out_specs=[
            pl.BlockSpec((GBLK, 128), row),
            pl.BlockSpec((GBLK, 128), row),
            pl.BlockSpec((GBLK, 128), row),
        ],
        out_shape=[
            jax.ShapeDtypeStruct((GEP_H, 128), _f32),
            jax.ShapeDtypeStruct((GEP_H, 128), _f32),
            jax.ShapeDtypeStruct((GEP_H, 128), _f32),
        ],
    )(hvf, hvt, he0, aij, w1f, w1t, w1e, w1a, w2, w3, w4p1, w4p2, w4e,
      b123, b4, g16)


def _bus_mlp(hv, he1, u, bi, w1h, w1e, w1u, w1b, w2, w3, w4, b):
    x = (
        _mm(hv, w1h)
        + _mm(he1, w1e)
        + _mm(u, w1u)
        + _mm(bi, w1b)
        + b[0:1, :]
    )
    h = jnp.tanh(x)
    h = jnp.tanh(_mm(h, w2) + b[1:2, :])
    h = jnp.tanh(_mm(h, w3) + b[2:3, :])
    return _mm(h, w4) + b[3:4, :]


def _bus_body(hv, he1, u, bi, acc0, acc1, acc2, acc3,
              vw1h, vw1e, vw1u, vw1b, vw2, vw3, vw4, vb,
              ew1h, ew1e, ew1u, ew1b, ew2, ew3, ew4, eb,
              ow1h, ow1e, ow1u, ow1b, ow2, ow3, ow4, ob,
              g16, hvn, he1n, un):
    hv_ = hv[...]
    he1_ = he1[...]
    u_ = u[...]
    bi_ = bi[...]
    g = g16[...]
    d1 = _bus_mlp(hv_, he1_, u_, bi_, vw1h[...], vw1e[...], vw1u[...],
                  vw1b[...], vw2[...], vw3[...], vw4[...], vb)
    hvnew = _gnorm(hv_ + DT * d1 + (acc0[...] + acc1[...])
                   + (acc2[...] + acc3[...]), g)
    d2 = _bus_mlp(hvnew, he1_, u_, bi_, ew1h[...], ew1e[...], ew1u[...],
                  ew1b[...], ew2[...], ew3[...], ew4[...], eb)
    he1new = _gnorm(he1_ + DT * d2, g)
    d3 = _bus_mlp(hvnew, he1new, u_, bi_, ow1h[...], ow1e[...], ow1u[...],
                  ow1b[...], ow2[...], ow3[...], ow4[...], ob)
    hvn[...] = hvnew
    he1n[...] = he1new
    un[...] = u_ + DT * d3


def _tc_bus(hv, he1, u, bi, accs4, wsets, g16):
    grid = (NBG // BBLKG,)
    row = lambda i: (i, 0)
    full = lambda i: (0, 0)
    wspecs = []
    wargs = []
    for ws in wsets:
        wspecs += [pl.BlockSpec((128, 128), full)] * 7 + [
            pl.BlockSpec((8, 128), full)]
        wargs += list(ws)
    return pl.pallas_call(
        _bus_body,
        grid=grid,
        in_specs=[pl.BlockSpec((BBLKG, 128), row)] * 8 + wspecs
        + [pl.BlockSpec((128, 128), full)],
        out_specs=[pl.BlockSpec((BBLKG, 128), row)] * 3,
        out_shape=[
            jax.ShapeDtypeStruct((NBG, 128), _f32),
            jax.ShapeDtypeStruct((NBG, 128), _f32),
            jax.ShapeDtypeStruct((NBG, 128), _f32),
        ],
    )(hv, he1, u, bi, *accs4, *wargs, g16)


def _pack_edge_weights(params):
    eye8 = jnp.eye(8, dtype=_f32)
    bf = jnp.bfloat16

    def base16(rows, sl):
        out = jnp.zeros((16, 32), _f32)
        for i, name in enumerate(("pv00", "pv01", "pe0")):
            W1 = params[name][0][0]
            out = out.at[0:rows, i * 10:i * 10 + 10].set(W1[sl])
        return out

    w1f = base16(10, slice(1, 11))
    w1t = base16(10, slice(11, 21))
    w1e = base16(10, slice(21, 31))
    w1a = base16(8, slice(31, 39))

    def blockdiag(layer):
        out = jnp.zeros((32, 32), _f32)
        for i, name in enumerate(("pv00", "pv01", "pe0")):
            out = out.at[i * 10:i * 10 + 10, i * 10:i * 10 + 10].set(
                params[name][layer][0])
        return out

    w2 = blockdiag(1)
    w3 = blockdiag(2)

    def w4base(which):
        out = jnp.zeros((32, 16), _f32)
        i = ("pv00", "pv01", "pe0").index(which)
        return out.at[i * 10:i * 10 + 10, 0:10].set(params[which][3][0])

    big = lambda m: jnp.kron(eye8, m).astype(bf)
    ws = (big(w1f), big(w1t), big(w1e), big(w1a), big(w2), big(w3),
          big(w4base("pv00")), big(w4base("pv01")), big(w4base("pe0")))

    def bvec(layer):
        out = jnp.zeros((32,), _f32)
        for i, name in enumerate(("pv00", "pv01", "pe0")):
            out = out.at[i * 10:i * 10 + 10].set(params[name][layer][1])
        return out

    trow = jnp.zeros((32,), _f32)
    for i, name in enumerate(("pv00", "pv01", "pe0")):
        trow = trow.at[i * 10:i * 10 + 10].set(params[name][0][0][0])
    b1 = bvec(0)
    b2 = bvec(1)
    b3 = bvec(2)
    b4p1 = jnp.zeros((16,), _f32).at[0:10].set(params["pv00"][3][1])
    b4p2 = jnp.zeros((16,), _f32).at[0:10].set(params["pv01"][3][1])
    b4e = jnp.zeros((16,), _f32).at[0:10].set(params["pe0"][3][1])
    b4 = jnp.concatenate([
        jnp.stack([jnp.tile(b4p1, 8), jnp.tile(b4p2, 8), jnp.tile(b4e, 8)]),
        jnp.zeros((5, 128), _f32),
    ])
    return ws, (trow, b1, b2, b3), b4


def _edge_bias(trow, b1, b2, b3, t):
    rows = jnp.stack([jnp.tile(b1 + t * trow, 8), jnp.tile(b2, 8),
                      jnp.tile(b3, 8)])
    return jnp.concatenate([rows, jnp.zeros((5, 256), _f32)])


def _pack_bus_weights(p, d_out):
    eye8 = jnp.eye(8, dtype=_f32)
    bf = jnp.bfloat16
    W1 = p[0][0]

    def base(rows, sl):
        return jnp.zeros((16, 16), _f32).at[0:rows, 0:10].set(W1[sl])

    w1h = base(10, slice(1, 11))
    w1e = base(10, slice(11, 21))
    w1u = base(2, slice(21, 23))
    w1b = base(10, slice(23, 33))
    w2 = jnp.zeros((16, 16), _f32).at[0:10, 0:10].set(p[1][0])
    w3 = jnp.zeros((16, 16), _f32).at[0:10, 0:10].set(p[2][0])
    w4 = jnp.zeros((16, 16), _f32).at[0:10, 0:d_out].set(p[3][0])
    big = lambda m: jnp.kron(eye8, m).astype(bf)
    ws = (big(w1h), big(w1e), big(w1u), big(w1b), big(w2), big(w3), big(w4))
    pad16 = lambda v, n: jnp.zeros((16,), _f32).at[0:n].set(v)
    trow = pad16(W1[0], 10)
    bs = (pad16(p[0][1], 10), pad16(p[1][1], 10), pad16(p[2][1], 10),
          pad16(p[3][1], d_out))
    return ws, (trow, bs)


def _bus_bias(trow, bs, t):
    rows = jnp.stack([jnp.tile(bs[0] + t * trow, 8), jnp.tile(bs[1], 8),
                      jnp.tile(bs[2], 8), jnp.tile(bs[3], 8)])
    return jnp.concatenate([rows, jnp.zeros((4, 128), _f32)])


def kernel(A_flat, B_flat, A0, params):
    a_ij = A_flat.reshape(B * E, 8)
    a_ij = jnp.concatenate([a_ij, jnp.zeros((B * E, 8), _f32)], axis=1)
    a_ij = jnp.concatenate([a_ij, jnp.zeros((EP - ETOT, 16), _f32)], axis=0)
    a_ij = a_ij.reshape(GEP, 128)
    aijA, aijB = a_ij[:GEP_H], a_ij[GEP_H:]
    b_i = B_flat.reshape(M, 10)
    b_i = jnp.concatenate(
        [
            jnp.concatenate([b_i, jnp.zeros((M, 6), _f32)], axis=1),
            jnp.zeros((NB - M, 16), _f32),
        ],
        axis=0,
    ).reshape(NBG, 128)

    boff = (jnp.arange(B, dtype=jnp.int32) * NBUS)[:, None]
    idxf = (A0[:, :, 0].astype(jnp.int32) + boff).reshape(ETOT)
    idxt = (A0[:, :, 1].astype(jnp.int32) + boff).reshape(ETOT)
    pad = jnp.full((EP - ETOT,), DUMMY, jnp.int32)
    idxf2 = jnp.concatenate([idxf, pad]).reshape(IDXROWS // CH, CH * 128)
    idxt2 = jnp.concatenate([idxt, pad]).reshape(IDXROWS // CH, CH * 128)
    nh = IDXR_H // CH
    idxfA, idxfB = idxf2[:nh], idxf2[nh:]
    idxtA, idxtB = idxt2[:nh], idxt2[nh:]

    ew, ebp, eb4 = _pack_edge_weights(params)
    vw, vbp = _pack_bus_weights(params["pv10"], 10)
    pw, pbp = _pack_bus_weights(params["pe1"], 10)
    ow, obp = _pack_bus_weights(params["pout1"], 2)
    g16 = jnp.kron(jnp.eye(8, dtype=_f32), jnp.ones((16, 16), _f32))

    hv = jnp.zeros((NB, 16), _f32)
    he0A = jnp.zeros((GEP_H, 128), _f32)
    he0B = jnp.zeros((GEP_H, 128), _f32)
    he1 = jnp.zeros((NBG, 128), _f32)
    u = jnp.zeros((NBG, 128), _f32).at[:, 0::16].set(1.0)
    zeros_nb = jnp.zeros((NB, 16), _f32)
    sc_gather, sc_scatter = _sc_kernels()

    for step in range(NSTEP):
        t = DT * step
        ebias = _edge_bias(*ebp, t)
        vbias = _bus_bias(*vbp, t)
        pbias = _bus_bias(*pbp, t)
        obias = _bus_bias(*obp, t)

        hvfA, hvtA = sc_gather(hv, idxfA, idxtA)
        hvfB, hvtB = sc_gather(hv, idxfB, idxtB)
        p1A, p2A, he0A = _tc_edge(hvfA.reshape(GEP_H, 128),
                                  hvtA.reshape(GEP_H, 128),
                                  he0A, aijA, ew, ebias, eb4, g16)
        accsA = sc_scatter(p1A.reshape(EP_H, 16), idxfA,
                           p2A.reshape(EP_H, 16), idxtA, zeros_nb)
        p1B, p2B, he0B = _tc_edge(hvfB.reshape(GEP_H, 128),
                                  hvtB.reshape(GEP_H, 128),
                                  he0B, aijB, ew, ebias, eb4, g16)
        accsB = sc_scatter(p1B.reshape(EP_H, 16), idxfB,
                           p2B.reshape(EP_H, 16), idxtB, zeros_nb)
        accs4 = [accsA[:NB].reshape(NBG, 128), accsA[NB:].reshape(NBG, 128),
                 accsB[:NB].reshape(NBG, 128), accsB[NB:].reshape(NBG, 128)]
        hvg, he1, u = _tc_bus(
            hv.reshape(NBG, 128), he1, u, b_i, accs4,
            [(*vw, vbias), (*pw, pbias), (*ow, obias)], g16,
        )
        hv = hvg.reshape(NB, 16)

    return u.reshape(NB, 16)[:M, 0:2].reshape(B, NBUS, 2)


# speedup vs baseline: 1.3889x; 1.1287x over previous
import functools

import jax
import jax.numpy as jnp
from jax import lax
from jax.experimental import pallas as pl
from jax.experimental.pallas import tpu as pltpu
from jax.experimental.pallas import tpu_sc as plsc

B = 4
NBUS = 10000
E = 160000
LAT = 10
DT = 0.2
NSTEP = 5

M = B * NBUS
NB = 40960
DUMMY = M
ETOT = B * E
EP = 655360
IDXROWS = EP // 128
NWORK = 32
RPT = IDXROWS // NWORK
ROWS_PER_TILE = NB // 16

GEP = EP // 8
NBG = NB // 8
GBLK = 512
BBLKG = 512

_f32 = jnp.float32


CH = 16
NCH = RPT // CH


@functools.lru_cache(maxsize=None)
def _sc_kernels():
    mesh = plsc.VectorSubcoreMesh(core_axis_name="c", subcore_axis_name="s")

    @functools.partial(
        pl.kernel,
        out_type=(
            jax.ShapeDtypeStruct((EP, 16), _f32),
            jax.ShapeDtypeStruct((EP, 16), _f32),
        ),
        mesh=mesh,
        scratch_types=[
            pltpu.VMEM((NCH, CH * 128), jnp.int32),
            pltpu.VMEM((2, CH * 128, 16), _f32),
            pltpu.VMEM_SHARED((NB, 16), _f32),
            pltpu.SemaphoreType.DMA,
            pltpu.SemaphoreType.DMA,
        ],
        compiler_params=pltpu.CompilerParams(use_tc_tiling_on_sc=False),
    )
    def sc_gather(tab, idxf, idxt, outf, outt, idx_v, buf, tabsh, sem0, sem1):
        c = lax.axis_index("c")
        sub = lax.axis_index("s")
        wid = sub * 2 + c
        base = wid * RPT * 128
        sems = (sem0, sem1)
        pltpu.sync_copy(
            tab.at[pl.ds(sub * ROWS_PER_TILE, ROWS_PER_TILE)],
            tabsh.at[pl.ds(sub * ROWS_PER_TILE, ROWS_PER_TILE)],
        )
        plsc.subcore_barrier()

        def run(idx_hbm, out_hbm):
            pltpu.sync_copy(idx_hbm.at[pl.ds(wid * NCH, NCH)], idx_v)
            d = {0: pltpu.async_copy(
                tabsh.at[idx_v.at[0]], buf.at[0], sems[0])}
            for k in range(NCH):
                par = k & 1
                if k + 1 < NCH:
                    d[k + 1] = pltpu.async_copy(
                        tabsh.at[idx_v.at[k + 1]],
                        buf.at[1 - par], sems[1 - par])
                d[k].wait()
                pltpu.sync_copy(buf.at[par],
                                out_hbm.at[pl.ds(base + k * CH * 128,
                                                 CH * 128)])

        run(idxf, outf)
        run(idxt, outt)

    @functools.partial(
        pl.kernel,
        out_type=jax.ShapeDtypeStruct((2 * NB, 16), _f32),
        mesh=mesh,
        scratch_types=[
            pltpu.VMEM((NCH, CH * 128), jnp.int32),
            pltpu.VMEM((2, CH * 128, 16), _f32),
            pltpu.VMEM_SHARED((NB, 16), _f32),
            pltpu.SemaphoreType.DMA,
            pltpu.SemaphoreType.DMA,
            pltpu.SemaphoreType.DMA,
        ],
        compiler_params=pltpu.CompilerParams(use_tc_tiling_on_sc=False),
    )
    def sc_scatter(p1, idxf, p2, idxt, zeros_hbm, out,
                   idx_v, val, acc, lsem0, lsem1, asem):
        c = lax.axis_index("c")
        s = lax.axis_index("s")
        wid = s * 2 + c
        base = wid * RPT * 128
        lsems = (lsem0, lsem1)
        pltpu.sync_copy(
            zeros_hbm.at[pl.ds(s * ROWS_PER_TILE, ROWS_PER_TILE)],
            acc.at[pl.ds(s * ROWS_PER_TILE, ROWS_PER_TILE)],
        )
        plsc.subcore_barrier()

        def run(p_hbm, idx_hbm):
            pltpu.sync_copy(idx_hbm.at[pl.ds(wid * NCH, NCH)], idx_v)
            d = {0: pltpu.async_copy(
                p_hbm.at[pl.ds(base, CH * 128)], val.at[0], lsems[0])}
            for k in range(NCH):
                par = k & 1
                if k + 1 < NCH:
                    d[k + 1] = pltpu.async_copy(
                        p_hbm.at[pl.ds(base + (k + 1) * CH * 128, CH * 128)],
                        val.at[1 - par], lsems[1 - par])
                d[k].wait()
                pltpu.async_copy(
                    val.at[par], acc.at[idx_v.at[k]],
                    asem, add=True).wait()

        run(p1, idxf)
        run(p2, idxt)
        plsc.subcore_barrier()
        pltpu.sync_copy(
            acc.at[pl.ds(s * ROWS_PER_TILE, ROWS_PER_TILE)],
            out.at[pl.ds(c * NB + s * ROWS_PER_TILE, ROWS_PER_TILE)],
        )

    return sc_gather, sc_scatter


def _mm(a, b):
    return jnp.dot(a.astype(jnp.bfloat16), b, preferred_element_type=_f32)


def _gnorm(v, g16):
    s = jnp.dot(v * v, g16, preferred_element_type=_f32)
    return v / (jnp.sqrt(s) + 1.0)


def _edge_body(hvf, hvt, he0, aij, w1f, w1t, w1e, w1a, w2, w3,
               w4p1, w4p2, w4e, b123, b4, g16, p1, p2, he0n):
    x = (
        _mm(hvf[...], w1f[...])
        + _mm(hvt[...], w1t[...])
        + _mm(he0[...], w1e[...])
        + _mm(aij[...], w1a[...])
        + b123[0:1, :]
    )
    h = jnp.tanh(x)
    h = jnp.tanh(_mm(h, w2[...]) + b123[1:2, :])
    h = jnp.tanh(_mm(h, w3[...]) + b123[2:3, :])
    p1[...] = _mm(h, w4p1[...]) + b4[0:1, :]
    p2[...] = _mm(h, w4p2[...]) + b4[1:2, :]
    d = _mm(h, w4e[...]) + b4[2:3, :]
    he0n[...] = _gnorm(he0[...] + DT * d, g16[...])


def _tc_edge(hvf, hvt, he0, aij, ew, b123, b4, g16):
    grid = (GEP // GBLK,)
    row = lambda i: (i, 0)
    full = lambda i: (0, 0)
    w1f, w1t, w1e, w1a, w2, w3, w4p1, w4p2, w4e = ew
    return pl.pallas_call(
        _edge_body,
        grid=grid,
        in_specs=[
            pl.BlockSpec((GBLK, 128), row),
            pl.BlockSpec((GBLK, 128), row),
            pl.BlockSpec((GBLK, 128), row),
            pl.BlockSpec((GBLK, 128), row),
            pl.BlockSpec((128, 256), full),
            pl.BlockSpec((128, 256), full),
            pl.BlockSpec((128, 256), full),
            pl.BlockSpec((128, 256), full),
            pl.BlockSpec((256, 256), full),
            pl.BlockSpec((256, 256), full),
            pl.BlockSpec((256, 128), full),
            pl.BlockSpec((256, 128), full),
            pl.BlockSpec((256, 128), full),
            pl.BlockSpec((8, 256), full),
            pl.BlockSpec((8, 128), full),
            pl.BlockSpec((128, 128), full),
        ],
        out_specs=[
            pl.BlockSpec((GBLK, 128), row),
            pl.BlockSpec((GBLK, 128), row),
            pl.BlockSpec((GBLK, 128), row),
        ],
        out_shape=[
            jax.ShapeDtypeStruct((GEP, 128), _f32),
            jax.ShapeDtypeStruct((GEP, 128), _f32),
            jax.ShapeDtypeStruct((GEP, 128), _f32),
        ],
    )(hvf, hvt, he0, aij, w1f, w1t, w1e, w1a, w2, w3, w4p1, w4p2, w4e,
      b123, b4, g16)


def _bus_mlp(hv, he1, u, bi, w1h, w1e, w1u, w1b, w2, w3, w4, b):
    x = (
        _mm(hv, w1h)
        + _mm(he1, w1e)
        + _mm(u, w1u)
        + _mm(bi, w1b)
        + b[0:1, :]
    )
    h = jnp.tanh(x)
    h = jnp.tanh(_mm(h, w2) + b[1:2, :])
    h = jnp.tanh(_mm(h, w3) + b[2:3, :])
    return _mm(h, w4) + b[3:4, :]


def _bus_body(hv, he1, u, bi, acc0, acc1,
              vw1h, vw1e, vw1u, vw1b, vw2, vw3, vw4, vb,
              ew1h, ew1e, ew1u, ew1b, ew2, ew3, ew4, eb,
              ow1h, ow1e, ow1u, ow1b, ow2, ow3, ow4, ob,
              g16, hvn, he1n, un):
    hv_ = hv[...]
    he1_ = he1[...]
    u_ = u[...]
    bi_ = bi[...]
    g = g16[...]
    d1 = _bus_mlp(hv_, he1_, u_, bi_, vw1h[...], vw1e[...], vw1u[...],
                  vw1b[...], vw2[...], vw3[...], vw4[...], vb)
    hvnew = _gnorm(hv_ + DT * d1 + acc0[...] + acc1[...], g)
    d2 = _bus_mlp(hvnew, he1_, u_, bi_, ew1h[...], ew1e[...], ew1u[...],
                  ew1b[...], ew2[...], ew3[...], ew4[...], eb)
    he1new = _gnorm(he1_ + DT * d2, g)
    d3 = _bus_mlp(hvnew, he1new, u_, bi_, ow1h[...], ow1e[...], ow1u[...],
                  ow1b[...], ow2[...], ow3[...], ow4[...], ob)
    hvn[...] = hvnew
    he1n[...] = he1new
    un[...] = u_ + DT * d3


def _tc_bus(hv, he1, u, bi, acc0, acc1, wsets, g16):
    grid = (NBG // BBLKG,)
    row = lambda i: (i, 0)
    full = lambda i: (0, 0)
    wspecs = []
    wargs = []
    for ws in wsets:
        wspecs += [pl.BlockSpec((128, 128), full)] * 7 + [
            pl.BlockSpec((8, 128), full)]
        wargs += list(ws)
    return pl.pallas_call(
        _bus_body,
        grid=grid,
        in_specs=[pl.BlockSpec((BBLKG, 128), row)] * 6 + wspecs
        + [pl.BlockSpec((128, 128), full)],
        out_specs=[pl.BlockSpec((BBLKG, 128), row)] * 3,
        out_shape=[
            jax.ShapeDtypeStruct((NBG, 128), _f32),
            jax.ShapeDtypeStruct((NBG, 128), _f32),
            jax.ShapeDtypeStruct((NBG, 128), _f32),
        ],
    )(hv, he1, u, bi, acc0, acc1, *wargs, g16)


def _pack_edge_weights(params):
    eye8 = jnp.eye(8, dtype=_f32)
    bf = jnp.bfloat16

    def base16(rows, sl):
        out = jnp.zeros((16, 32), _f32)
        for i, name in enumerate(("pv00", "pv01", "pe0")):
            W1 = params[name][0][0]
            out = out.at[0:rows, i * 10:i * 10 + 10].set(W1[sl])
        return out

    w1f = base16(10, slice(1, 11))
    w1t = base16(10, slice(11, 21))
    w1e = base16(10, slice(21, 31))
    w1a = base16(8, slice(31, 39))

    def blockdiag(layer):
        out = jnp.zeros((32, 32), _f32)
        for i, name in enumerate(("pv00", "pv01", "pe0")):
            out = out.at[i * 10:i * 10 + 10, i * 10:i * 10 + 10].set(
                params[name][layer][0])
        return out

    w2 = blockdiag(1)
    w3 = blockdiag(2)

    def w4base(which):
        out = jnp.zeros((32, 16), _f32)
        i = ("pv00", "pv01", "pe0").index(which)
        return out.at[i * 10:i * 10 + 10, 0:10].set(params[which][3][0])

    big = lambda m: jnp.kron(eye8, m).astype(bf)
    ws = (big(w1f), big(w1t), big(w1e), big(w1a), big(w2), big(w3),
          big(w4base("pv00")), big(w4base("pv01")), big(w4base("pe0")))

    def bvec(layer):
        out = jnp.zeros((32,), _f32)
        for i, name in enumerate(("pv00", "pv01", "pe0")):
            out = out.at[i * 10:i * 10 + 10].set(params[name][layer][1])
        return out

    trow = jnp.zeros((32,), _f32)
    for i, name in enumerate(("pv00", "pv01", "pe0")):
        trow = trow.at[i * 10:i * 10 + 10].set(params[name][0][0][0])
    b1 = bvec(0)
    b2 = bvec(1)
    b3 = bvec(2)
    b4p1 = jnp.zeros((16,), _f32).at[0:10].set(params["pv00"][3][1])
    b4p2 = jnp.zeros((16,), _f32).at[0:10].set(params["pv01"][3][1])
    b4e = jnp.zeros((16,), _f32).at[0:10].set(params["pe0"][3][1])
    b4 = jnp.concatenate([
        jnp.stack([jnp.tile(b4p1, 8), jnp.tile(b4p2, 8), jnp.tile(b4e, 8)]),
        jnp.zeros((5, 128), _f32),
    ])
    return ws, (trow, b1, b2, b3), b4


def _edge_bias(trow, b1, b2, b3, t):
    rows = jnp.stack([jnp.tile(b1 + t * trow, 8), jnp.tile(b2, 8),
                      jnp.tile(b3, 8)])
    return jnp.concatenate([rows, jnp.zeros((5, 256), _f32)])


def _pack_bus_weights(p, d_out):
    eye8 = jnp.eye(8, dtype=_f32)
    bf = jnp.bfloat16
    W1 = p[0][0]

    def base(rows, sl):
        return jnp.zeros((16, 16), _f32).at[0:rows, 0:10].set(W1[sl])

    w1h = base(10, slice(1, 11))
    w1e = base(10, slice(11, 21))
    w1u = base(2, slice(21, 23))
    w1b = base(10, slice(23, 33))
    w2 = jnp.zeros((16, 16), _f32).at[0:10, 0:10].set(p[1][0])
    w3 = jnp.zeros((16, 16), _f32).at[0:10, 0:10].set(p[2][0])
    w4 = jnp.zeros((16, 16), _f32).at[0:10, 0:d_out].set(p[3][0])
    big = lambda m: jnp.kron(eye8, m).astype(bf)
    ws = (big(w1h), big(w1e), big(w1u), big(w1b), big(w2), big(w3), big(w4))
    pad16 = lambda v, n: jnp.zeros((16,), _f32).at[0:n].set(v)
    trow = pad16(W1[0], 10)
    bs = (pad16(p[0][1], 10), pad16(p[1][1], 10), pad16(p[2][1], 10),
          pad16(p[3][1], d_out))
    return ws, (trow, bs)


def _bus_bias(trow, bs, t):
    rows = jnp.stack([jnp.tile(bs[0] + t * trow, 8), jnp.tile(bs[1], 8),
                      jnp.tile(bs[2], 8), jnp.tile(bs[3], 8)])
    return jnp.concatenate([rows, jnp.zeros((4, 128), _f32)])


def kernel(A_flat, B_flat, A0, params):
    a_ij = A_flat.reshape(B * E, 8)
    a_ij = jnp.concatenate([a_ij, jnp.zeros((B * E, 8), _f32)], axis=1)
    a_ij = jnp.concatenate([a_ij, jnp.zeros((EP - ETOT, 16), _f32)], axis=0)
    a_ij = a_ij.reshape(GEP, 128)
    b_i = B_flat.reshape(M, 10)
    b_i = jnp.concatenate(
        [
            jnp.concatenate([b_i, jnp.zeros((M, 6), _f32)], axis=1),
            jnp.zeros((NB - M, 16), _f32),
        ],
        axis=0,
    ).reshape(NBG, 128)

    boff = (jnp.arange(B, dtype=jnp.int32) * NBUS)[:, None]
    idxf = (A0[:, :, 0].astype(jnp.int32) + boff).reshape(ETOT)
    idxt = (A0[:, :, 1].astype(jnp.int32) + boff).reshape(ETOT)
    pad = jnp.full((EP - ETOT,), DUMMY, jnp.int32)
    idxf2 = jnp.concatenate([idxf, pad]).reshape(IDXROWS // CH, CH * 128)
    idxt2 = jnp.concatenate([idxt, pad]).reshape(IDXROWS // CH, CH * 128)

    ew, ebp, eb4 = _pack_edge_weights(params)
    vw, vbp = _pack_bus_weights(params["pv10"], 10)
    pw, pbp = _pack_bus_weights(params["pe1"], 10)
    ow, obp = _pack_bus_weights(params["pout1"], 2)
    g16 = jnp.kron(jnp.eye(8, dtype=_f32), jnp.ones((16, 16), _f32))

    hv = jnp.zeros((NB, 16), _f32)
    he0 = jnp.zeros((GEP, 128), _f32)
    he1 = jnp.zeros((NBG, 128), _f32)
    u = jnp.zeros((NBG, 128), _f32).at[:, 0::16].set(1.0)
    zeros_nb = jnp.zeros((NB, 16), _f32)
    sc_gather, sc_scatter = _sc_kernels()

    for step in range(NSTEP):
        t = DT * step
        ebias = _edge_bias(*ebp, t)
        vbias = _bus_bias(*vbp, t)
        pbias = _bus_bias(*pbp, t)
        obias = _bus_bias(*obp, t)

        hvf, hvt = sc_gather(hv, idxf2, idxt2)
        p1, p2, he0 = _tc_edge(hvf.reshape(GEP, 128), hvt.reshape(GEP, 128),
                               he0, a_ij, ew, ebias, eb4, g16)
        accs = sc_scatter(p1.reshape(EP, 16), idxf2,
                          p2.reshape(EP, 16), idxt2, zeros_nb)
        acc0 = accs[:NB].reshape(NBG, 128)
        acc1 = accs[NB:].reshape(NBG, 128)
        hvg, he1, u = _tc_bus(
            hv.reshape(NBG, 128), he1, u, b_i, acc0, acc1,
            [(*vw, vbias), (*pw, pbias), (*ow, obias)], g16,
        )
        hv = hvg.reshape(NB, 16)

    return u.reshape(NB, 16)[:M, 0:2].reshape(B, NBUS, 2)


# speedup vs baseline: 1.5005x; 1.0804x over previous
import functools

import jax
import jax.numpy as jnp
from jax import lax
from jax.experimental import pallas as pl
from jax.experimental.pallas import tpu as pltpu
from jax.experimental.pallas import tpu_sc as plsc

B = 4
NBUS = 10000
E = 160000
LAT = 10
DT = 0.2
NSTEP = 5

M = B * NBUS
NB = 40960
DUMMY = M
ETOT = B * E
EP = 655360
IDXROWS = EP // 128
NWORK = 32
RPT = IDXROWS // NWORK
ROWS_PER_TILE = NB // 16

GEP = EP // 8
NBG = NB // 8
GBLK = 1024
BBLKG = 1024

_f32 = jnp.float32


CH = 16
NCH = RPT // CH


@functools.lru_cache(maxsize=None)
def _sc_kernels():
    mesh = plsc.VectorSubcoreMesh(core_axis_name="c", subcore_axis_name="s")

    @functools.partial(
        pl.kernel,
        out_type=(
            jax.ShapeDtypeStruct((EP, 16), _f32),
            jax.ShapeDtypeStruct((EP, 16), _f32),
        ),
        mesh=mesh,
        scratch_types=[
            pltpu.VMEM((NCH, CH * 128), jnp.int32),
            pltpu.VMEM((2, CH * 128, 16), _f32),
            pltpu.VMEM_SHARED((NB, 16), _f32),
            pltpu.SemaphoreType.DMA,
            pltpu.SemaphoreType.DMA,
        ],
        compiler_params=pltpu.CompilerParams(use_tc_tiling_on_sc=False),
    )
    def sc_gather(tab, idxf, idxt, outf, outt, idx_v, buf, tabsh, sem0, sem1):
        c = lax.axis_index("c")
        sub = lax.axis_index("s")
        wid = sub * 2 + c
        base = wid * RPT * 128
        sems = (sem0, sem1)
        pltpu.sync_copy(
            tab.at[pl.ds(sub * ROWS_PER_TILE, ROWS_PER_TILE)],
            tabsh.at[pl.ds(sub * ROWS_PER_TILE, ROWS_PER_TILE)],
        )
        plsc.subcore_barrier()

        def run(idx_hbm, out_hbm):
            pltpu.sync_copy(idx_hbm.at[pl.ds(wid * NCH, NCH)], idx_v)
            d = {0: pltpu.async_copy(
                tabsh.at[idx_v.at[0]], buf.at[0], sems[0])}
            for k in range(NCH):
                par = k & 1
                if k + 1 < NCH:
                    d[k + 1] = pltpu.async_copy(
                        tabsh.at[idx_v.at[k + 1]],
                        buf.at[1 - par], sems[1 - par])
                d[k].wait()
                pltpu.sync_copy(buf.at[par],
                                out_hbm.at[pl.ds(base + k * CH * 128,
                                                 CH * 128)])

        run(idxf, outf)
        run(idxt, outt)

    @functools.partial(
        pl.kernel,
        out_type=jax.ShapeDtypeStruct((2 * NB, 16), _f32),
        mesh=mesh,
        scratch_types=[
            pltpu.VMEM((NCH, CH * 128), jnp.int32),
            pltpu.VMEM((2, CH * 128, 16), _f32),
            pltpu.VMEM_SHARED((NB, 16), _f32),
            pltpu.SemaphoreType.DMA,
            pltpu.SemaphoreType.DMA,
            pltpu.SemaphoreType.DMA,
        ],
        compiler_params=pltpu.CompilerParams(use_tc_tiling_on_sc=False),
    )
    def sc_scatter(p1, idxf, p2, idxt, zeros_hbm, out,
                   idx_v, val, acc, lsem0, lsem1, asem):
        c = lax.axis_index("c")
        s = lax.axis_index("s")
        wid = s * 2 + c
        base = wid * RPT * 128
        lsems = (lsem0, lsem1)
        pltpu.sync_copy(
            zeros_hbm.at[pl.ds(s * ROWS_PER_TILE, ROWS_PER_TILE)],
            acc.at[pl.ds(s * ROWS_PER_TILE, ROWS_PER_TILE)],
        )
        plsc.subcore_barrier()

        def run(p_hbm, idx_hbm):
            pltpu.sync_copy(idx_hbm.at[pl.ds(wid * NCH, NCH)], idx_v)
            d = {0: pltpu.async_copy(
                p_hbm.at[pl.ds(base, CH * 128)], val.at[0], lsems[0])}
            for k in range(NCH):
                par = k & 1
                if k + 1 < NCH:
                    d[k + 1] = pltpu.async_copy(
                        p_hbm.at[pl.ds(base + (k + 1) * CH * 128, CH * 128)],
                        val.at[1 - par], lsems[1 - par])
                d[k].wait()
                pltpu.async_copy(
                    val.at[par], acc.at[idx_v.at[k]],
                    asem, add=True).wait()

        run(p1, idxf)
        run(p2, idxt)
        plsc.subcore_barrier()
        pltpu.sync_copy(
            acc.at[pl.ds(s * ROWS_PER_TILE, ROWS_PER_TILE)],
            out.at[pl.ds(c * NB + s * ROWS_PER_TILE, ROWS_PER_TILE)],
        )

    return sc_gather, sc_scatter


def _mm(a, b):
    return jnp.dot(a.astype(jnp.bfloat16), b, preferred_element_type=_f32)


def _gnorm(v, g16):
    s = jnp.dot(v * v, g16, preferred_element_type=_f32)
    return v / (jnp.sqrt(s) + 1.0)


def _edge_body(hvf, hvt, he0, aij, w1f, w1t, w1e, w1a, w2, w3,
               w4p1, w4p2, w4e, b123, b4, g16, p1, p2, he0n):
    x = (
        _mm(hvf[...], w1f[...])
        + _mm(hvt[...], w1t[...])
        + _mm(he0[...], w1e[...])
        + _mm(aij[...], w1a[...])
        + b123[0:1, :]
    )
    h = jnp.tanh(x)
    h = jnp.tanh(_mm(h, w2[...]) + b123[1:2, :])
    h = jnp.tanh(_mm(h, w3[...]) + b123[2:3, :])
    p1[...] = _mm(h, w4p1[...]) + b4[0:1, :]
    p2[...] = _mm(h, w4p2[...]) + b4[1:2, :]
    d = _mm(h, w4e[...]) + b4[2:3, :]
    he0n[...] = _gnorm(he0[...] + DT * d, g16[...])


def _tc_edge(hvf, hvt, he0, aij, ew, b123, b4, g16):
    grid = (GEP // GBLK,)
    row = lambda i: (i, 0)
    full = lambda i: (0, 0)
    w1f, w1t, w1e, w1a, w2, w3, w4p1, w4p2, w4e = ew
    return pl.pallas_call(
        _edge_body,
        grid=grid,
        in_specs=[
            pl.BlockSpec((GBLK, 128), row),
            pl.BlockSpec((GBLK, 128), row),
            pl.BlockSpec((GBLK, 128), row),
            pl.BlockSpec((GBLK, 128), row),
            pl.BlockSpec((128, 256), full),
            pl.BlockSpec((128, 256), full),
            pl.BlockSpec((128, 256), full),
            pl.BlockSpec((128, 256), full),
            pl.BlockSpec((256, 256), full),
            pl.BlockSpec((256, 256), full),
            pl.BlockSpec((256, 128), full),
            pl.BlockSpec((256, 128), full),
            pl.BlockSpec((256, 128), full),
            pl.BlockSpec((8, 256), full),
            pl.BlockSpec((8, 128), full),
            pl.BlockSpec((128, 128), full),
        ],
        out_specs=[
            pl.BlockSpec((GBLK, 128), row),
            pl.BlockSpec((GBLK, 128), row),
            pl.BlockSpec((GBLK, 128), row),
        ],
        out_shape=[
            jax.ShapeDtypeStruct((GEP, 128), _f32),
            jax.ShapeDtypeStruct((GEP, 128), _f32),
            jax.ShapeDtypeStruct((GEP, 128), _f32),
        ],
    )(hvf, hvt, he0, aij, w1f, w1t, w1e, w1a, w2, w3, w4p1, w4p2, w4e,
      b123, b4, g16)


def _bus_mlp(hv, he1, u, bi, w1h, w1e, w1u, w1b, w2, w3, w4, b):
    x = (
        _mm(hv, w1h)
        + _mm(he1, w1e)
        + _mm(u, w1u)
        + _mm(bi, w1b)
        + b[0:1, :]
    )
    h = jnp.tanh(x)
    h = jnp.tanh(_mm(h, w2) + b[1:2, :])
    h = jnp.tanh(_mm(h, w3) + b[2:3, :])
    return _mm(h, w4) + b[3:4, :]


def _bus_body(hv, he1, u, bi, acc0, acc1,
              vw1h, vw1e, vw1u, vw1b, vw2, vw3, vw4, vb,
              ew1h, ew1e, ew1u, ew1b, ew2, ew3, ew4, eb,
              ow1h, ow1e, ow1u, ow1b, ow2, ow3, ow4, ob,
              g16, hvn, he1n, un):
    hv_ = hv[...]
    he1_ = he1[...]
    u_ = u[...]
    bi_ = bi[...]
    g = g16[...]
    d1 = _bus_mlp(hv_, he1_, u_, bi_, vw1h[...], vw1e[...], vw1u[...],
                  vw1b[...], vw2[...], vw3[...], vw4[...], vb)
    hvnew = _gnorm(hv_ + DT * d1 + acc0[...] + acc1[...], g)
    d2 = _bus_mlp(hvnew, he1_, u_, bi_, ew1h[...], ew1e[...], ew1u[...],
                  ew1b[...], ew2[...], ew3[...], ew4[...], eb)
    he1new = _gnorm(he1_ + DT * d2, g)
    d3 = _bus_mlp(hvnew, he1new, u_, bi_, ow1h[...], ow1e[...], ow1u[...],
                  ow1b[...], ow2[...], ow3[...], ow4[...], ob)
    hvn[...] = hvnew
    he1n[...] = he1new
    un[...] = u_ + DT * d3


def _tc_bus(hv, he1, u, bi, acc0, acc1, wsets, g16):
    grid = (NBG // BBLKG,)
    row = lambda i: (i, 0)
    full = lambda i: (0, 0)
    wspecs = []
    wargs = []
    for ws in wsets:
        wspecs += [pl.BlockSpec((128, 128), full)] * 7 + [
            pl.BlockSpec((8, 128), full)]
        wargs += list(ws)
    return pl.pallas_call(
        _bus_body,
        grid=grid,
        in_specs=[pl.BlockSpec((BBLKG, 128), row)] * 6 + wspecs
        + [pl.BlockSpec((128, 128), full)],
        out_specs=[pl.BlockSpec((BBLKG, 128), row)] * 3,
        out_shape=[
            jax.ShapeDtypeStruct((NBG, 128), _f32),
            jax.ShapeDtypeStruct((NBG, 128), _f32),
            jax.ShapeDtypeStruct((NBG, 128), _f32),
        ],
    )(hv, he1, u, bi, acc0, acc1, *wargs, g16)


def _pack_edge_weights(params):
    eye8 = jnp.eye(8, dtype=_f32)
    bf = jnp.bfloat16

    def base16(rows, sl):
        out = jnp.zeros((16, 32), _f32)
        for i, name in enumerate(("pv00", "pv01", "pe0")):
            W1 = params[name][0][0]
            out = out.at[0:rows, i * 10:i * 10 + 10].set(W1[sl])
        return out

    w1f = base16(10, slice(1, 11))
    w1t = base16(10, slice(11, 21))
    w1e = base16(10, slice(21, 31))
    w1a = base16(8, slice(31, 39))

    def blockdiag(layer):
        out = jnp.zeros((32, 32), _f32)
        for i, name in enumerate(("pv00", "pv01", "pe0")):
            out = out.at[i * 10:i * 10 + 10, i * 10:i * 10 + 10].set(
                params[name][layer][0])
        return out

    w2 = blockdiag(1)
    w3 = blockdiag(2)

    def w4base(which):
        out = jnp.zeros((32, 16), _f32)
        i = ("pv00", "pv01", "pe0").index(which)
        return out.at[i * 10:i * 10 + 10, 0:10].set(params[which][3][0])

    big = lambda m: jnp.kron(eye8, m).astype(bf)
    ws = (big(w1f), big(w1t), big(w1e), big(w1a), big(w2), big(w3),
          big(w4base("pv00")), big(w4base("pv01")), big(w4base("pe0")))

    def bvec(layer):
        out = jnp.zeros((32,), _f32)
        for i, name in enumerate(("pv00", "pv01", "pe0")):
            out = out.at[i * 10:i * 10 + 10].set(params[name][layer][1])
        return out

    trow = jnp.zeros((32,), _f32)
    for i, name in enumerate(("pv00", "pv01", "pe0")):
        trow = trow.at[i * 10:i * 10 + 10].set(params[name][0][0][0])
    b1 = bvec(0)
    b2 = bvec(1)
    b3 = bvec(2)
    b4p1 = jnp.zeros((16,), _f32).at[0:10].set(params["pv00"][3][1])
    b4p2 = jnp.zeros((16,), _f32).at[0:10].set(params["pv01"][3][1])
    b4e = jnp.zeros((16,), _f32).at[0:10].set(params["pe0"][3][1])
    b4 = jnp.concatenate([
        jnp.stack([jnp.tile(b4p1, 8), jnp.tile(b4p2, 8), jnp.tile(b4e, 8)]),
        jnp.zeros((5, 128), _f32),
    ])
    return ws, (trow, b1, b2, b3), b4


def _edge_bias(trow, b1, b2, b3, t):
    rows = jnp.stack([jnp.tile(b1 + t * trow, 8), jnp.tile(b2, 8),
                      jnp.tile(b3, 8)])
    return jnp.concatenate([rows, jnp.zeros((5, 256), _f32)])


def _pack_bus_weights(p, d_out):
    eye8 = jnp.eye(8, dtype=_f32)
    bf = jnp.bfloat16
    W1 = p[0][0]

    def base(rows, sl):
        return jnp.zeros((16, 16), _f32).at[0:rows, 0:10].set(W1[sl])

    w1h = base(10, slice(1, 11))
    w1e = base(10, slice(11, 21))
    w1u = base(2, slice(21, 23))
    w1b = base(10, slice(23, 33))
    w2 = jnp.zeros((16, 16), _f32).at[0:10, 0:10].set(p[1][0])
    w3 = jnp.zeros((16, 16), _f32).at[0:10, 0:10].set(p[2][0])
    w4 = jnp.zeros((16, 16), _f32).at[0:10, 0:d_out].set(p[3][0])
    big = lambda m: jnp.kron(eye8, m).astype(bf)
    ws = (big(w1h), big(w1e), big(w1u), big(w1b), big(w2), big(w3), big(w4))
    pad16 = lambda v, n: jnp.zeros((16,), _f32).at[0:n].set(v)
    trow = pad16(W1[0], 10)
    bs = (pad16(p[0][1], 10), pad16(p[1][1], 10), pad16(p[2][1], 10),
          pad16(p[3][1], d_out))
    return ws, (trow, bs)


def _bus_bias(trow, bs, t):
    rows = jnp.stack([jnp.tile(bs[0] + t * trow, 8), jnp.tile(bs[1], 8),
                      jnp.tile(bs[2], 8), jnp.tile(bs[3], 8)])
    return jnp.concatenate([rows, jnp.zeros((4, 128), _f32)])


def kernel(A_flat, B_flat, A0, params):
    a_ij = A_flat.reshape(B * E, 8)
    a_ij = jnp.concatenate([a_ij, jnp.zeros((B * E, 8), _f32)], axis=1)
    a_ij = jnp.concatenate([a_ij, jnp.zeros((EP - ETOT, 16), _f32)], axis=0)
    a_ij = a_ij.reshape(GEP, 128)
    b_i = B_flat.reshape(M, 10)
    b_i = jnp.concatenate(
        [
            jnp.concatenate([b_i, jnp.zeros((M, 6), _f32)], axis=1),
            jnp.zeros((NB - M, 16), _f32),
        ],
        axis=0,
    ).reshape(NBG, 128)

    boff = (jnp.arange(B, dtype=jnp.int32) * NBUS)[:, None]
    idxf = (A0[:, :, 0].astype(jnp.int32) + boff).reshape(ETOT)
    idxt = (A0[:, :, 1].astype(jnp.int32) + boff).reshape(ETOT)
    pad = jnp.full((EP - ETOT,), DUMMY, jnp.int32)
    idxf2 = jnp.concatenate([idxf, pad]).reshape(IDXROWS // CH, CH * 128)
    idxt2 = jnp.concatenate([idxt, pad]).reshape(IDXROWS // CH, CH * 128)

    ew, ebp, eb4 = _pack_edge_weights(params)
    vw, vbp = _pack_bus_weights(params["pv10"], 10)
    pw, pbp = _pack_bus_weights(params["pe1"], 10)
    ow, obp = _pack_bus_weights(params["pout1"], 2)
    g16 = jnp.kron(jnp.eye(8, dtype=_f32), jnp.ones((16, 16), _f32))

    hv = jnp.zeros((NB, 16), _f32)
    he0 = jnp.zeros((GEP, 128), _f32)
    he1 = jnp.zeros((NBG, 128), _f32)
    u = jnp.zeros((NBG, 128), _f32).at[:, 0::16].set(1.0)
    zeros_nb = jnp.zeros((NB, 16), _f32)
    sc_gather, sc_scatter = _sc_kernels()

    for step in range(NSTEP):
        t = DT * step
        ebias = _edge_bias(*ebp, t)
        vbias = _bus_bias(*vbp, t)
        pbias = _bus_bias(*pbp, t)
        obias = _bus_bias(*obp, t)

        hvf, hvt = sc_gather(hv, idxf2, idxt2)
        p1, p2, he0 = _tc_edge(hvf.reshape(GEP, 128), hvt.reshape(GEP, 128),
                               he0, a_ij, ew, ebias, eb4, g16)
        accs = sc_scatter(p1.reshape(EP, 16), idxf2,
                          p2.reshape(EP, 16), idxt2, zeros_nb)
        acc0 = accs[:NB].reshape(NBG, 128)
        acc1 = accs[NB:].reshape(NBG, 128)
        hvg, he1, u = _tc_bus(
            hv.reshape(NBG, 128), he1, u, b_i, acc0, acc1,
            [(*vw, vbias), (*pw, pbias), (*ow, obias)], g16,
        )
        hv = hvg.reshape(NB, 16)

    return u.reshape(NB, 16)[:M, 0:2].reshape(B, NBUS, 2)


# speedup vs baseline: 1.5276x; 1.0180x over previous
import functools

import jax
import jax.numpy as jnp
from jax import lax
from jax.experimental import pallas as pl
from jax.experimental.pallas import tpu as pltpu
from jax.experimental.pallas import tpu_sc as plsc

B = 4
NBUS = 10000
E = 160000
LAT = 10
DT = 0.2
NSTEP = 5

M = B * NBUS
NB = 40960
DUMMY = M
ETOT = B * E
EP = 655360
IDXROWS = EP // 128
NWORK = 32
RPT = IDXROWS // NWORK
ROWS_PER_TILE = NB // 16

GEP = EP // 8
NBG = NB // 8
GBLK = 2048
BBLKG = 1024

_f32 = jnp.float32


CH = 16
NCH = RPT // CH


@functools.lru_cache(maxsize=None)
def _sc_kernels():
    mesh = plsc.VectorSubcoreMesh(core_axis_name="c", subcore_axis_name="s")

    @functools.partial(
        pl.kernel,
        out_type=(
            jax.ShapeDtypeStruct((EP, 16), _f32),
            jax.ShapeDtypeStruct((EP, 16), _f32),
        ),
        mesh=mesh,
        scratch_types=[
            pltpu.VMEM((NCH, CH * 128), jnp.int32),
            pltpu.VMEM((2, CH * 128, 16), _f32),
            pltpu.VMEM_SHARED((NB, 16), _f32),
            pltpu.SemaphoreType.DMA,
            pltpu.SemaphoreType.DMA,
        ],
        compiler_params=pltpu.CompilerParams(use_tc_tiling_on_sc=False),
    )
    def sc_gather(tab, idxf, idxt, outf, outt, idx_v, buf, tabsh, sem0, sem1):
        c = lax.axis_index("c")
        sub = lax.axis_index("s")
        wid = sub * 2 + c
        base = wid * RPT * 128
        sems = (sem0, sem1)
        pltpu.sync_copy(
            tab.at[pl.ds(sub * ROWS_PER_TILE, ROWS_PER_TILE)],
            tabsh.at[pl.ds(sub * ROWS_PER_TILE, ROWS_PER_TILE)],
        )
        plsc.subcore_barrier()

        def run(idx_hbm, out_hbm):
            pltpu.sync_copy(idx_hbm.at[pl.ds(wid * NCH, NCH)], idx_v)
            d = {0: pltpu.async_copy(
                tabsh.at[idx_v.at[0]], buf.at[0], sems[0])}
            for k in range(NCH):
                par = k & 1
                if k + 1 < NCH:
                    d[k + 1] = pltpu.async_copy(
                        tabsh.at[idx_v.at[k + 1]],
                        buf.at[1 - par], sems[1 - par])
                d[k].wait()
                pltpu.sync_copy(buf.at[par],
                                out_hbm.at[pl.ds(base + k * CH * 128,
                                                 CH * 128)])

        run(idxf, outf)
        run(idxt, outt)

    @functools.partial(
        pl.kernel,
        out_type=jax.ShapeDtypeStruct((2 * NB, 16), _f32),
        mesh=mesh,
        scratch_types=[
            pltpu.VMEM((NCH, CH * 128), jnp.int32),
            pltpu.VMEM((2, CH * 128, 16), _f32),
            pltpu.VMEM_SHARED((NB, 16), _f32),
            pltpu.SemaphoreType.DMA,
            pltpu.SemaphoreType.DMA,
            pltpu.SemaphoreType.DMA,
        ],
        compiler_params=pltpu.CompilerParams(use_tc_tiling_on_sc=False),
    )
    def sc_scatter(p1, idxf, p2, idxt, zeros_hbm, out,
                   idx_v, val, acc, lsem0, lsem1, asem):
        c = lax.axis_index("c")
        s = lax.axis_index("s")
        wid = s * 2 + c
        base = wid * RPT * 128
        lsems = (lsem0, lsem1)
        pltpu.sync_copy(
            zeros_hbm.at[pl.ds(s * ROWS_PER_TILE, ROWS_PER_TILE)],
            acc.at[pl.ds(s * ROWS_PER_TILE, ROWS_PER_TILE)],
        )
        plsc.subcore_barrier()

        def run(p_hbm, idx_hbm):
            pltpu.sync_copy(idx_hbm.at[pl.ds(wid * NCH, NCH)], idx_v)
            d = {0: pltpu.async_copy(
                p_hbm.at[pl.ds(base, CH * 128)], val.at[0], lsems[0])}
            for k in range(NCH):
                par = k & 1
                if k + 1 < NCH:
                    d[k + 1] = pltpu.async_copy(
                        p_hbm.at[pl.ds(base + (k + 1) * CH * 128, CH * 128)],
                        val.at[1 - par], lsems[1 - par])
                d[k].wait()
                pltpu.async_copy(
                    val.at[par], acc.at[idx_v.at[k]],
                    asem, add=True).wait()

        run(p1, idxf)
        run(p2, idxt)
        plsc.subcore_barrier()
        pltpu.sync_copy(
            acc.at[pl.ds(s * ROWS_PER_TILE, ROWS_PER_TILE)],
            out.at[pl.ds(c * NB + s * ROWS_PER_TILE, ROWS_PER_TILE)],
        )

    return sc_gather, sc_scatter


def _mm(a, b):
    return jnp.dot(a.astype(jnp.bfloat16), b, preferred_element_type=_f32)


def _gnorm(v, g16):
    s = jnp.dot(v * v, g16, preferred_element_type=_f32)
    return v / (jnp.sqrt(s) + 1.0)


def _edge_body(hvf, hvt, he0, aij, w1f, w1t, w1e, w1a, w2, w3,
               w4p1, w4p2, w4e, b123, b4, g16, p1, p2, he0n):
    x = (
        _mm(hvf[...], w1f[...])
        + _mm(hvt[...], w1t[...])
        + _mm(he0[...], w1e[...])
        + _mm(aij[...], w1a[...])
        + b123[0:1, :]
    )
    h = jnp.tanh(x)
    h = jnp.tanh(_mm(h, w2[...]) + b123[1:2, :])
    h = jnp.tanh(_mm(h, w3[...]) + b123[2:3, :])
    p1[...] = _mm(h, w4p1[...]) + b4[0:1, :]
    p2[...] = _mm(h, w4p2[...]) + b4[1:2, :]
    d = _mm(h, w4e[...]) + b4[2:3, :]
    he0n[...] = _gnorm(he0[...] + DT * d, g16[...])


def _tc_edge(hvf, hvt, he0, aij, ew, b123, b4, g16):
    grid = (GEP // GBLK,)
    row = lambda i: (i, 0)
    full = lambda i: (0, 0)
    w1f, w1t, w1e, w1a, w2, w3, w4p1, w4p2, w4e = ew
    return pl.pallas_call(
        _edge_body,
        grid=grid,
        in_specs=[
            pl.BlockSpec((GBLK, 128), row),
            pl.BlockSpec((GBLK, 128), row),
            pl.BlockSpec((GBLK, 128), row),
            pl.BlockSpec((GBLK, 128), row),
            pl.BlockSpec((128, 256), full),
            pl.BlockSpec((128, 256), full),
            pl.BlockSpec((128, 256), full),
            pl.BlockSpec((128, 256), full),
            pl.BlockSpec((256, 256), full),
            pl.BlockSpec((256, 256), full),
            pl.BlockSpec((256, 128), full),
            pl.BlockSpec((256, 128), full),
            pl.BlockSpec((256, 128), full),
            pl.BlockSpec((8, 256), full),
            pl.BlockSpec((8, 128), full),
            pl.BlockSpec((128, 128), full),
        ],
        out_specs=[
            pl.BlockSpec((GBLK, 128), row),
            pl.BlockSpec((GBLK, 128), row),
            pl.BlockSpec((GBLK, 128), row),
        ],
        out_shape=[
            jax.ShapeDtypeStruct((GEP, 128), _f32),
            jax.ShapeDtypeStruct((GEP, 128), _f32),
            jax.ShapeDtypeStruct((GEP, 128), _f32),
        ],
    )(hvf, hvt, he0, aij, w1f, w1t, w1e, w1a, w2, w3, w4p1, w4p2, w4e,
      b123, b4, g16)


def _bus_mlp(hv, he1, u, bi, w1h, w1e, w1u, w1b, w2, w3, w4, b):
    x = (
        _mm(hv, w1h)
        + _mm(he1, w1e)
        + _mm(u, w1u)
        + _mm(bi, w1b)
        + b[0:1, :]
    )
    h = jnp.tanh(x)
    h = jnp.tanh(_mm(h, w2) + b[1:2, :])
    h = jnp.tanh(_mm(h, w3) + b[2:3, :])
    return _mm(h, w4) + b[3:4, :]


def _bus_body(hv, he1, u, bi, acc0, acc1,
              vw1h, vw1e, vw1u, vw1b, vw2, vw3, vw4, vb,
              ew1h, ew1e, ew1u, ew1b, ew2, ew3, ew4, eb,
              ow1h, ow1e, ow1u, ow1b, ow2, ow3, ow4, ob,
              g16, hvn, he1n, un):
    hv_ = hv[...]
    he1_ = he1[...]
    u_ = u[...]
    bi_ = bi[...]
    g = g16[...]
    d1 = _bus_mlp(hv_, he1_, u_, bi_, vw1h[...], vw1e[...], vw1u[...],
                  vw1b[...], vw2[...], vw3[...], vw4[...], vb)
    hvnew = _gnorm(hv_ + DT * d1 + acc0[...] + acc1[...], g)
    d2 = _bus_mlp(hvnew, he1_, u_, bi_, ew1h[...], ew1e[...], ew1u[...],
                  ew1b[...], ew2[...], ew3[...], ew4[...], eb)
    he1new = _gnorm(he1_ + DT * d2, g)
    d3 = _bus_mlp(hvnew, he1new, u_, bi_, ow1h[...], ow1e[...], ow1u[...],
                  ow1b[...], ow2[...], ow3[...], ow4[...], ob)
    hvn[...] = hvnew
    he1n[...] = he1new
    un[...] = u_ + DT * d3


def _tc_bus(hv, he1, u, bi, acc0, acc1, wsets, g16):
    grid = (NBG // BBLKG,)
    row = lambda i: (i, 0)
    full = lambda i: (0, 0)
    wspecs = []
    wargs = []
    for ws in wsets:
        wspecs += [pl.BlockSpec((128, 128), full)] * 7 + [
            pl.BlockSpec((8, 128), full)]
        wargs += list(ws)
    return pl.pallas_call(
        _bus_body,
        grid=grid,
        in_specs=[pl.BlockSpec((BBLKG, 128), row)] * 6 + wspecs
        + [pl.BlockSpec((128, 128), full)],
        out_specs=[pl.BlockSpec((BBLKG, 128), row)] * 3,
        out_shape=[
            jax.ShapeDtypeStruct((NBG, 128), _f32),
            jax.ShapeDtypeStruct((NBG, 128), _f32),
            jax.ShapeDtypeStruct((NBG, 128), _f32),
        ],
    )(hv, he1, u, bi, acc0, acc1, *wargs, g16)


def _pack_edge_weights(params):
    eye8 = jnp.eye(8, dtype=_f32)
    bf = jnp.bfloat16

    def base16(rows, sl):
        out = jnp.zeros((16, 32), _f32)
        for i, name in enumerate(("pv00", "pv01", "pe0")):
            W1 = params[name][0][0]
            out = out.at[0:rows, i * 10:i * 10 + 10].set(W1[sl])
        return out

    w1f = base16(10, slice(1, 11))
    w1t = base16(10, slice(11, 21))
    w1e = base16(10, slice(21, 31))
    w1a = base16(8, slice(31, 39))

    def blockdiag(layer):
        out = jnp.zeros((32, 32), _f32)
        for i, name in enumerate(("pv00", "pv01", "pe0")):
            out = out.at[i * 10:i * 10 + 10, i * 10:i * 10 + 10].set(
                params[name][layer][0])
        return out

    w2 = blockdiag(1)
    w3 = blockdiag(2)

    def w4base(which):
        out = jnp.zeros((32, 16), _f32)
        i = ("pv00", "pv01", "pe0").index(which)
        return out.at[i * 10:i * 10 + 10, 0:10].set(params[which][3][0])

    big = lambda m: jnp.kron(eye8, m).astype(bf)
    ws = (big(w1f), big(w1t), big(w1e), big(w1a), big(w2), big(w3),
          big(w4base("pv00")), big(w4base("pv01")), big(w4base("pe0")))

    def bvec(layer):
        out = jnp.zeros((32,), _f32)
        for i, name in enumerate(("pv00", "pv01", "pe0")):
            out = out.at[i * 10:i * 10 + 10].set(params[name][layer][1])
        return out

    trow = jnp.zeros((32,), _f32)
    for i, name in enumerate(("pv00", "pv01", "pe0")):
        trow = trow.at[i * 10:i * 10 + 10].set(params[name][0][0][0])
    b1 = bvec(0)
    b2 = bvec(1)
    b3 = bvec(2)
    b4p1 = jnp.zeros((16,), _f32).at[0:10].set(params["pv00"][3][1])
    b4p2 = jnp.zeros((16,), _f32).at[0:10].set(params["pv01"][3][1])
    b4e = jnp.zeros((16,), _f32).at[0:10].set(params["pe0"][3][1])
    b4 = jnp.concatenate([
        jnp.stack([jnp.tile(b4p1, 8), jnp.tile(b4p2, 8), jnp.tile(b4e, 8)]),
        jnp.zeros((5, 128), _f32),
    ])
    return ws, (trow, b1, b2, b3), b4


def _edge_bias(trow, b1, b2, b3, t):
    rows = jnp.stack([jnp.tile(b1 + t * trow, 8), jnp.tile(b2, 8),
                      jnp.tile(b3, 8)])
    return jnp.concatenate([rows, jnp.zeros((5, 256), _f32)])


def _pack_bus_weights(p, d_out):
    eye8 = jnp.eye(8, dtype=_f32)
    bf = jnp.bfloat16
    W1 = p[0][0]

    def base(rows, sl):
        return jnp.zeros((16, 16), _f32).at[0:rows, 0:10].set(W1[sl])

    w1h = base(10, slice(1, 11))
    w1e = base(10, slice(11, 21))
    w1u = base(2, slice(21, 23))
    w1b = base(10, slice(23, 33))
    w2 = jnp.zeros((16, 16), _f32).at[0:10, 0:10].set(p[1][0])
    w3 = jnp.zeros((16, 16), _f32).at[0:10, 0:10].set(p[2][0])
    w4 = jnp.zeros((16, 16), _f32).at[0:10, 0:d_out].set(p[3][0])
    big = lambda m: jnp.kron(eye8, m).astype(bf)
    ws = (big(w1h), big(w1e), big(w1u), big(w1b), big(w2), big(w3), big(w4))
    pad16 = lambda v, n: jnp.zeros((16,), _f32).at[0:n].set(v)
    trow = pad16(W1[0], 10)
    bs = (pad16(p[0][1], 10), pad16(p[1][1], 10), pad16(p[2][1], 10),
          pad16(p[3][1], d_out))
    return ws, (trow, bs)


def _bus_bias(trow, bs, t):
    rows = jnp.stack([jnp.tile(bs[0] + t * trow, 8), jnp.tile(bs[1], 8),
                      jnp.tile(bs[2], 8), jnp.tile(bs[3], 8)])
    return jnp.concatenate([rows, jnp.zeros((4, 128), _f32)])


def kernel(A_flat, B_flat, A0, params):
    a_ij = A_flat.reshape(B * E, 8)
    a_ij = jnp.concatenate([a_ij, jnp.zeros((B * E, 8), _f32)], axis=1)
    a_ij = jnp.concatenate([a_ij, jnp.zeros((EP - ETOT, 16), _f32)], axis=0)
    a_ij = a_ij.reshape(GEP, 128)
    b_i = B_flat.reshape(M, 10)
    b_i = jnp.concatenate(
        [
            jnp.concatenate([b_i, jnp.zeros((M, 6), _f32)], axis=1),
            jnp.zeros((NB - M, 16), _f32),
        ],
        axis=0,
    ).reshape(NBG, 128)

    boff = (jnp.arange(B, dtype=jnp.int32) * NBUS)[:, None]
    idxf = (A0[:, :, 0].astype(jnp.int32) + boff).reshape(ETOT)
    idxt = (A0[:, :, 1].astype(jnp.int32) + boff).reshape(ETOT)
    pad = jnp.full((EP - ETOT,), DUMMY, jnp.int32)
    idxf2 = jnp.concatenate([idxf, pad]).reshape(IDXROWS // CH, CH * 128)
    idxt2 = jnp.concatenate([idxt, pad]).reshape(IDXROWS // CH, CH * 128)

    ew, ebp, eb4 = _pack_edge_weights(params)
    vw, vbp = _pack_bus_weights(params["pv10"], 10)
    pw, pbp = _pack_bus_weights(params["pe1"], 10)
    ow, obp = _pack_bus_weights(params["pout1"], 2)
    g16 = jnp.kron(jnp.eye(8, dtype=_f32), jnp.ones((16, 16), _f32))

    hv = jnp.zeros((NB, 16), _f32)
    he0 = jnp.zeros((GEP, 128), _f32)
    he1 = jnp.zeros((NBG, 128), _f32)
    u = jnp.zeros((NBG, 128), _f32).at[:, 0::16].set(1.0)
    zeros_nb = jnp.zeros((NB, 16), _f32)
    sc_gather, sc_scatter = _sc_kernels()

    for step in range(NSTEP):
        t = DT * step
        ebias = _edge_bias(*ebp, t)
        vbias = _bus_bias(*vbp, t)
        pbias = _bus_bias(*pbp, t)
        obias = _bus_bias(*obp, t)

        hvf, hvt = sc_gather(hv, idxf2, idxt2)
        p1, p2, he0 = _tc_edge(hvf.reshape(GEP, 128), hvt.reshape(GEP, 128),
                               he0, a_ij, ew, ebias, eb4, g16)
        accs = sc_scatter(p1.reshape(EP, 16), idxf2,
                          p2.reshape(EP, 16), idxt2, zeros_nb)
        acc0 = accs[:NB].reshape(NBG, 128)
        acc1 = accs[NB:].reshape(NBG, 128)
        hvg, he1, u = _tc_bus(
            hv.reshape(NBG, 128), he1, u, b_i, acc0, acc1,
            [(*vw, vbias), (*pw, pbias), (*ow, obias)], g16,
        )
        hv = hvg.reshape(NB, 16)

    return u.reshape(NB, 16)[:M, 0:2].reshape(B, NBUS, 2)
